# Initial kernel scaffold; baseline (speedup 1.0000x reference)
import jax
import jax.numpy as jnp
from jax.experimental import pallas as pl


def kernel(h, x, edge_index, mask_ligand, edge_attr, params):
    raise NotImplementedError("write your pallas kernel here")


# speedup vs baseline: 1.1986x; 1.1986x over previous
import math
import jax
import jax.numpy as jnp
from jax.experimental import pallas as pl

N = 10000
E = 320000
HIDDEN = 128
EDGE_FEAT_DIM = 4
NUM_RBF = 16
N_HEADS = 16
HEAD_DIM = HIDDEN // N_HEADS
CUTOFF = 10.0
SCALE = 1.0 / math.sqrt(HEAD_DIM)


def _layer_norm(v, g, b):
    mu = jnp.mean(v, axis=-1, keepdims=True)
    var = jnp.mean((v - mu) ** 2, axis=-1, keepdims=True)
    return (v - mu) / jnp.sqrt(var + 1e-5) * g + b


def kernel(h, x, edge_index, mask_ligand, edge_attr, params):
    p = params
    src, dst = edge_index[0], edge_index[1]
    num_edges = src.shape[0]
    rel = x[dst] - x[src]
    d2 = jnp.sum(rel * rel, axis=-1, keepdims=True)
    d = jnp.sqrt(d2 + 1e-8)
    offset = jnp.linspace(0.0, CUTOFF, NUM_RBF)
    coeff = -0.5 / (offset[1] - offset[0]) ** 2
    d_feat = jnp.exp(coeff * (d - offset[None, :]) ** 2)
    edge_feat = (d_feat[:, :, None] * edge_attr[:, None, :]).reshape(num_edges, -1)
    kv_in = jnp.concatenate([h[src], edge_feat], axis=-1)
    t = kv_in @ p["kv_w1"].T + p["kv_b1"]
    t = jax.nn.relu(_layer_norm(t, p["kv_ln_g"], p["kv_ln_b"]))
    kv = t @ p["kv_w2"].T + p["kv_b2"]
    k_, v_ = jnp.split(kv, 2, axis=-1)
    q = (h[dst] @ p["q_w"].T + p["q_b"]).reshape(num_edges, N_HEADS, HEAD_DIM)
    k_ = k_.reshape(num_edges, N_HEADS, HEAD_DIM)
    v_ = v_.reshape(num_edges, N_HEADS, HEAD_DIM)
    attn_logits = jnp.sum(q * k_, axis=-1) * SCALE
    tb = edge_feat @ p["ab_w1"].T + p["ab_b1"]
    tb = jax.nn.relu(_layer_norm(tb, p["ab_ln_g"], p["ab_ln_b"]))
    attn_bias = tb @ p["ab_w2"].T + p["ab_b2"]
    attn_logits = attn_logits + attn_bias
    ex = jnp.exp(attn_logits)
    s = jax.ops.segment_sum(ex, dst, num_segments=N)
    attn_weights = ex / (s[dst] + 1e-16)
    msg = (attn_weights[:, :, None] * v_).reshape(num_edges, HIDDEN)
    agg = jax.ops.segment_sum(msg, dst, num_segments=N)
    agg = agg @ p["out_w"].T + p["out_b"]
    tn = jnp.concatenate([agg, h], axis=-1) @ p["nm_w1"].T + p["nm_b1"]
    tn = jax.nn.relu(_layer_norm(tn, p["nm_ln_g"], p["nm_ln_b"]))
    h_new = h + (tn @ p["nm_w2"].T + p["nm_b2"])
    coef = jnp.tanh(jax.nn.silu(msg @ p["x_w1"].T + p["x_b1"]) @ p["x_w2"].T)
    vec = rel / (d + 1.0) * coef
    dx = jax.ops.segment_sum(vec, dst, num_segments=N)
    x_new = x + dx * mask_ligand.astype(x.dtype)[:, None]
    return (h_new, x_new)


# speedup vs baseline: 3.1885x; 2.6602x over previous
import functools
import math

import jax
import jax.numpy as jnp
from jax import lax
from jax.experimental import pallas as pl
from jax.experimental.pallas import tpu as pltpu
from jax.experimental.pallas import tpu_sc as plsc

N = 10000
E = 320000
HIDDEN = 128
EDGE_FEAT_DIM = 4
NUM_RBF = 16
N_HEADS = 16
HEAD_DIM = HIDDEN // N_HEADS
CUTOFF = 10.0
SCALE = 1.0 / math.sqrt(HEAD_DIM)
OUTER = NUM_RBF * EDGE_FEAT_DIM

TAB = 256
BN = 1000
BE = 2000

NC = 2
NS = 16
NW = NC * NS
EPW = E // NW
CCH = 400
NCH = EPW // CCH
WCH = 104
NWB = N // WCH
WTAIL = N - NWB * WCH

_f32 = jnp.float32


@functools.lru_cache(maxsize=1)
def _sc_mesh():
    return plsc.VectorSubcoreMesh(
        core_axis_name="c", subcore_axis_name="s",
        num_cores=NC, num_subcores=NS)


@functools.lru_cache(maxsize=1)
def _sc_mesh1():
    return plsc.VectorSubcoreMesh(
        core_axis_name="c", subcore_axis_name="s",
        num_cores=1, num_subcores=NS)


EPT = E // NS
SCCH = 200
NCHS = EPT // SCCH


def _ln(v, g, b):
    mu = jnp.mean(v, axis=1, keepdims=True)
    var = jnp.mean((v - mu) ** 2, axis=1, keepdims=True)
    return (v - mu) * jax.lax.rsqrt(var + 1e-5) * g + b


def _dot(a, b):
    return jnp.dot(a, b, preferred_element_type=_f32)


def _node_pre(h_ref, x_ref, qwT_ref, qb_ref, w1hT_ref, tabA_ref, tabB_ref):
    h = h_ref[...]
    hk = _dot(h, w1hT_ref[...])
    q = _dot(h, qwT_ref[...]) + qb_ref[...]
    xpad = jnp.concatenate(
        [x_ref[...], jnp.zeros((h.shape[0], TAB - HIDDEN - 3), _f32)], axis=1)
    tabA_ref[...] = jnp.concatenate([hk, xpad], axis=1)
    tabB_ref[...] = jnp.concatenate([q, xpad], axis=1)


def _run_node_pre(h, x, qwT, qb, w1hT):
    grid = (N // BN,)
    full = lambda shp: pl.BlockSpec(shp, lambda i: (0, 0))
    return pl.pallas_call(
        _node_pre,
        grid=grid,
        in_specs=[
            pl.BlockSpec((BN, HIDDEN), lambda i: (i, 0)),
            pl.BlockSpec((BN, 3), lambda i: (i, 0)),
            full((HIDDEN, HIDDEN)),
            full((1, HIDDEN)),
            full((HIDDEN, HIDDEN)),
        ],
        out_specs=[
            pl.BlockSpec((BN, TAB), lambda i: (i, 0)),
            pl.BlockSpec((BN, TAB), lambda i: (i, 0)),
        ],
        out_shape=[
            jax.ShapeDtypeStruct((N, TAB), _f32),
            jax.ShapeDtypeStruct((N, TAB), _f32),
        ],
    )(h, x, qwT, qb, w1hT)


def _edge_mlp(gA_ref, gB_ref, ea_ref, w1eT_ref, kvb1_ref, lng_ref, lnb_ref,
              w2T_ref, kvb2_ref, abw1T_ref, abb1_ref, ablng_ref, ablnb_ref,
              abw2T_ref, abb2_ref, combo_ref, v_ref):
    a = gA_ref[...]
    b = gB_ref[...]
    xs = a[:, HIDDEN:HIDDEN + 3]
    xd = b[:, HIDDEN:HIDDEN + 3]
    rel = xd - xs
    d2 = jnp.sum(rel * rel, axis=1, keepdims=True)
    d = jnp.sqrt(d2 + 1e-8)
    delta = CUTOFF / (NUM_RBF - 1)
    off = lax.broadcasted_iota(jnp.int32, (1, NUM_RBF), 1).astype(_f32) * delta
    df = jnp.exp((-0.5 / (delta * delta)) * (d - off) ** 2)
    ci = lax.broadcasted_iota(jnp.int32, (NUM_RBF, OUTER), 1)
    ri = lax.broadcasted_iota(jnp.int32, (NUM_RBF, OUTER), 0)
    Rm = (ci // EDGE_FEAT_DIM == ri).astype(_f32)
    cj = lax.broadcasted_iota(jnp.int32, (EDGE_FEAT_DIM, OUTER), 1)
    rj = lax.broadcasted_iota(jnp.int32, (EDGE_FEAT_DIM, OUTER), 0)
    Sm = (cj % EDGE_FEAT_DIM == rj).astype(_f32)
    ef = _dot(df, Rm) * _dot(ea_ref[...], Sm)
    t1 = a[:, :HIDDEN] + _dot(ef, w1eT_ref[...]) + kvb1_ref[...]
    t1 = jnp.maximum(_ln(t1, lng_ref[...], lnb_ref[...]), 0.0)
    kv = _dot(t1, w2T_ref[...]) + kvb2_ref[...]
    k = kv[:, :HIDDEN]
    v = kv[:, HIDDEN:]
    q = b[:, :HIDDEN]
    di = lax.broadcasted_iota(jnp.int32, (HIDDEN, N_HEADS), 0)
    hi = lax.broadcasted_iota(jnp.int32, (HIDDEN, N_HEADS), 1)
    Hm = (di // HEAD_DIM == hi).astype(_f32)
    logits = _dot(q * k, Hm) * SCALE
    tb = _dot(ef, abw1T_ref[...]) + abb1_ref[...]
    tb = jnp.maximum(_ln(tb, ablng_ref[...], ablnb_ref[...]), 0.0)
    ab = _dot(tb, abw2T_ref[...]) + abb2_ref[...]
    n = rel.shape[0]
    combo_ref[...] = jnp.concatenate(
        [jnp.exp(logits + ab), rel / (d + 1.0),
         jnp.zeros((n, HIDDEN - N_HEADS - 3), _f32)], axis=1)
    v_ref[...] = v


def _run_edge_mlp(gA, gB, ea, w1eT, kvb1, lng, lnb, w2T, kvb2,
                  abw1T, abb1, ablng, ablnb, abw2T, abb2):
    grid = (E // BE,)
    full = lambda shp: pl.BlockSpec(shp, lambda i: (0, 0))
    return pl.pallas_call(
        _edge_mlp,
        grid=grid,
        in_specs=[
            pl.BlockSpec((BE, TAB), lambda i: (i, 0)),
            pl.BlockSpec((BE, TAB), lambda i: (i, 0)),
            pl.BlockSpec((BE, EDGE_FEAT_DIM), lambda i: (i, 0)),
            full((OUTER, HIDDEN)),
            full((1, HIDDEN)),
            full((1, HIDDEN)),
            full((1, HIDDEN)),
            full((HIDDEN, 2 * HIDDEN)),
            full((1, 2 * HIDDEN)),
            full((OUTER, HIDDEN)),
            full((1, HIDDEN)),
            full((1, HIDDEN)),
            full((1, HIDDEN)),
            full((HIDDEN, N_HEADS)),
            full((1, N_HEADS)),
        ],
        out_specs=[
            pl.BlockSpec((BE, HIDDEN), lambda i: (i, 0)),
            pl.BlockSpec((BE, HIDDEN), lambda i: (i, 0)),
        ],
        out_shape=[
            jax.ShapeDtypeStruct((E, HIDDEN), _f32),
            jax.ShapeDtypeStruct((E, HIDDEN), _f32),
        ],
    )(gA, gB, ea, w1eT, kvb1, lng, lnb, w2T, kvb2,
      abw1T, abb1, ablng, ablnb, abw2T, abb2)


def _edge2(combo_ref, sg_ref, v_ref, xw1T_ref, xb1_ref, xw2_ref,
           msg_ref, vecp_ref):
    combo = combo_ref[...]
    w = combo[:, :N_HEADS] / (sg_ref[...][:, :N_HEADS] + 1e-16)
    relod = combo[:, N_HEADS:N_HEADS + 3]
    hi = lax.broadcasted_iota(jnp.int32, (N_HEADS, HIDDEN), 0)
    di = lax.broadcasted_iota(jnp.int32, (N_HEADS, HIDDEN), 1)
    Bm = (di // HEAD_DIM == hi).astype(_f32)
    msg = _dot(w, Bm) * v_ref[...]
    m1 = _dot(msg, xw1T_ref[...]) + xb1_ref[...]
    m1 = m1 * jax.nn.sigmoid(m1)
    cc = jnp.sum(m1 * xw2_ref[...], axis=1, keepdims=True)
    coef = jnp.tanh(cc)
    vec = relod * coef
    msg_ref[...] = msg
    vecp_ref[...] = jnp.concatenate(
        [vec, jnp.zeros((vec.shape[0], HIDDEN - 3), _f32)], axis=1)


def _run_edge2(combo, sg, v, xw1T, xb1, xw2):
    grid = (E // BE,)
    full = lambda shp: pl.BlockSpec(shp, lambda i: (0, 0))
    return pl.pallas_call(
        _edge2,
        grid=grid,
        in_specs=[
            pl.BlockSpec((BE, HIDDEN), lambda i: (i, 0)),
            pl.BlockSpec((BE, HIDDEN), lambda i: (i, 0)),
            pl.BlockSpec((BE, HIDDEN), lambda i: (i, 0)),
            full((HIDDEN, HIDDEN)),
            full((1, HIDDEN)),
            full((1, HIDDEN)),
        ],
        out_specs=[
            pl.BlockSpec((BE, HIDDEN), lambda i: (i, 0)),
            pl.BlockSpec((BE, HIDDEN), lambda i: (i, 0)),
        ],
        out_shape=[
            jax.ShapeDtypeStruct((E, HIDDEN), _f32),
            jax.ShapeDtypeStruct((E, HIDDEN), _f32),
        ],
    )(combo, sg, v, xw1T, xb1, xw2)


def _node_post(pa_ref, va_ref, h_ref, x_ref, mask_ref,
               outwT_ref, outb_ref, nmw1T_ref, nmb1_ref, nmlng_ref,
               nmlnb_ref, nmw2T_ref, nmb2_ref, hnew_ref, xnew_ref):
    agg = pa_ref[...]
    dxv = va_ref[...][:, :3]
    h = h_ref[...]
    aggo = _dot(agg, outwT_ref[...]) + outb_ref[...]
    tn = jnp.concatenate([aggo, h], axis=1)
    tn = _dot(tn, nmw1T_ref[...]) + nmb1_ref[...]
    tn = jnp.maximum(_ln(tn, nmlng_ref[...], nmlnb_ref[...]), 0.0)
    tn = _dot(tn, nmw2T_ref[...]) + nmb2_ref[...]
    hnew_ref[...] = h + tn
    xnew_ref[...] = x_ref[...] + dxv * mask_ref[...]


def _run_node_post(pm, pv, h, x, maskf, outwT, outb, nmw1T, nmb1,
                   nmlng, nmlnb, nmw2T, nmb2):
    grid = (N // BN,)
    full = lambda shp: pl.BlockSpec(shp, lambda i: (0, 0))
    return pl.pallas_call(
        _node_post,
        grid=grid,
        in_specs=[
            pl.BlockSpec((BN, HIDDEN), lambda i: (i, 0)),
            pl.BlockSpec((BN, HIDDEN), lambda i: (i, 0)),
            pl.BlockSpec((BN, HIDDEN), lambda i: (i, 0)),
            pl.BlockSpec((BN, 3), lambda i: (i, 0)),
            pl.BlockSpec((BN, 1), lambda i: (i, 0)),
            full((HIDDEN, HIDDEN)),
            full((1, HIDDEN)),
            full((2 * HIDDEN, HIDDEN)),
            full((1, HIDDEN)),
            full((1, HIDDEN)),
            full((1, HIDDEN)),
            full((HIDDEN, HIDDEN)),
            full((1, HIDDEN)),
        ],
        out_specs=[
            pl.BlockSpec((BN, HIDDEN), lambda i: (i, 0)),
            pl.BlockSpec((BN, 3), lambda i: (i, 0)),
        ],
        out_shape=[
            jax.ShapeDtypeStruct((N, HIDDEN), _f32),
            jax.ShapeDtypeStruct((N, 3), _f32),
        ],
    )(pm, pv, h, x, maskf, outwT, outb, nmw1T, nmb1,
      nmlng, nmlnb, nmw2T, nmb2)


def _sc_gather2_body(tabA, tabB, src, dst, gA, gB, idxv, bufv, sem):
    wid = lax.axis_index("s") * NC + lax.axis_index("c")
    base0 = wid * EPW

    def chunk(c, carry):
        base = base0 + c * CCH
        pltpu.sync_copy(src.at[pl.ds(base, CCH)], idxv)
        pltpu.async_copy(tabA.at[idxv], bufv, sem).wait()
        pltpu.sync_copy(bufv, gA.at[pl.ds(base, CCH), :])
        pltpu.sync_copy(dst.at[pl.ds(base, CCH)], idxv)
        pltpu.async_copy(tabB.at[idxv], bufv, sem).wait()
        pltpu.sync_copy(bufv, gB.at[pl.ds(base, CCH), :])
        return carry

    lax.fori_loop(0, NCH, chunk, 0)


def _run_sc_gather2(tabA, tabB, src, dst):
    return pl.kernel(
        _sc_gather2_body,
        out_type=[
            jax.ShapeDtypeStruct((E, TAB), _f32),
            jax.ShapeDtypeStruct((E, TAB), _f32),
        ],
        mesh=_sc_mesh(),
        scratch_types=[
            pltpu.VMEM((CCH,), jnp.int32),
            pltpu.VMEM((CCH, TAB), _f32),
            pltpu.SemaphoreType.DMA,
        ],
    )(tabA, tabB, src, dst)


def _sc_scatter_body(vals, dst, zrows, out, idxv, bufv, wbuf, tbuf, acc_ref):
    sid = lax.axis_index("s")
    for w in range(NWB // NS):
        r = (sid * (NWB // NS) + w) * WCH
        pltpu.sync_copy(zrows.at[pl.ds(r, WCH), :],
                        acc_ref.at[pl.ds(r, WCH), :])

    @pl.when(sid == 0)
    def _():
        pltpu.sync_copy(zrows.at[pl.ds(NWB * WCH, WTAIL), :],
                        acc_ref.at[pl.ds(NWB * WCH, WTAIL), :])

    plsc.subcore_barrier()
    base0 = sid * EPT

    def chunk(c, carry):
        base = base0 + c * SCCH
        pltpu.sync_copy(dst.at[pl.ds(base, SCCH)], idxv)
        pltpu.sync_copy(vals.at[pl.ds(base, SCCH), :], bufv)
        pltpu.sync_copy(bufv, acc_ref.at[idxv], add=True)
        return carry

    lax.fori_loop(0, NCHS, chunk, 0)
    plsc.subcore_barrier()
    for w in range(NWB // NS):
        r = (sid * (NWB // NS) + w) * WCH
        pltpu.sync_copy(acc_ref.at[pl.ds(r, WCH), :], wbuf)
        pltpu.sync_copy(wbuf, out.at[pl.ds(r, WCH), :])

    @pl.when(sid == 0)
    def _():
        pltpu.sync_copy(acc_ref.at[pl.ds(NWB * WCH, WTAIL), :], tbuf)
        pltpu.sync_copy(tbuf, out.at[pl.ds(NWB * WCH, WTAIL), :])


def _run_sc_scatter(vals, dst, zrows):
    return pl.kernel(
        _sc_scatter_body,
        out_type=jax.ShapeDtypeStruct((N, HIDDEN), _f32),
        mesh=_sc_mesh1(),
        scratch_types=[
            pltpu.VMEM((SCCH,), jnp.int32),
            pltpu.VMEM((SCCH, HIDDEN), _f32),
            pltpu.VMEM((WCH, HIDDEN), _f32),
            pltpu.VMEM((WTAIL, HIDDEN), _f32),
            pltpu.VMEM_SHARED((N, HIDDEN), _f32),
        ],
    )(vals, dst, zrows)


def _sc_gather1_body(tab, dst, out, idxv, bufv, sem):
    wid = lax.axis_index("s") * NC + lax.axis_index("c")
    base0 = wid * EPW

    def chunk(c, carry):
        base = base0 + c * CCH
        pltpu.sync_copy(dst.at[pl.ds(base, CCH)], idxv)
        pltpu.async_copy(tab.at[idxv], bufv, sem).wait()
        pltpu.sync_copy(bufv, out.at[pl.ds(base, CCH), :])
        return carry

    lax.fori_loop(0, NCH, chunk, 0)


def _run_sc_gather1(tab, dst):
    return pl.kernel(
        _sc_gather1_body,
        out_type=jax.ShapeDtypeStruct((E, HIDDEN), _f32),
        mesh=_sc_mesh(),
        scratch_types=[
            pltpu.VMEM((CCH,), jnp.int32),
            pltpu.VMEM((CCH, HIDDEN), _f32),
            pltpu.SemaphoreType.DMA,
        ],
    )(tab, dst)


def kernel(h, x, edge_index, mask_ligand, edge_attr, params):
    p = params
    row = lambda v: v.reshape(1, -1)
    qwT = p["q_w"].T
    w1hT = p["kv_w1"][:, :HIDDEN].T
    w1eT = p["kv_w1"][:, HIDDEN:].T
    w2T = p["kv_w2"].T
    abw1T = p["ab_w1"].T
    abw2T = p["ab_w2"].T
    outwT = p["out_w"].T
    nmw1T = p["nm_w1"].T
    nmw2T = p["nm_w2"].T
    xw1T = p["x_w1"].T
    maskf = mask_ligand.astype(_f32).reshape(N, 1)
    src = edge_index[0]
    dst = edge_index[1]

    tabA, tabB = _run_node_pre(h, x, qwT, row(p["q_b"]), w1hT)

    zrows = jnp.zeros((N, HIDDEN), _f32)
    gA, gB = _run_sc_gather2(tabA, tabB, src, dst)

    combo, v = _run_edge_mlp(
        gA, gB, edge_attr, w1eT, row(p["kv_b1"]), row(p["kv_ln_g"]),
        row(p["kv_ln_b"]), w2T, row(p["kv_b2"]), abw1T, row(p["ab_b1"]),
        row(p["ab_ln_g"]), row(p["ab_ln_b"]), abw2T, row(p["ab_b2"]))

    s = _run_sc_scatter(combo, dst, zrows)
    sg = _run_sc_gather1(s, dst)

    msg, vecp = _run_edge2(combo, sg, v, xw1T, row(p["x_b1"]),
                           p["x_w2"].reshape(1, HIDDEN))

    pm = _run_sc_scatter(msg, dst, zrows)
    pv = _run_sc_scatter(vecp, dst, zrows)

    h_new, x_new = _run_node_post(
        pm, pv, h, x, maskf, outwT, row(p["out_b"]), nmw1T, row(p["nm_b1"]),
        row(p["nm_ln_g"]), row(p["nm_ln_b"]), nmw2T, row(p["nm_b2"]))
    return (h_new, x_new)


# speedup vs baseline: 3.8727x; 1.2146x over previous
import functools
import math

import jax
import jax.numpy as jnp
from jax import lax
from jax.experimental import pallas as pl
from jax.experimental.pallas import tpu as pltpu
from jax.experimental.pallas import tpu_sc as plsc

N = 10000
E = 320000
HIDDEN = 128
EDGE_FEAT_DIM = 4
NUM_RBF = 16
N_HEADS = 16
HEAD_DIM = HIDDEN // N_HEADS
CUTOFF = 10.0
SCALE = 1.0 / math.sqrt(HEAD_DIM)
OUTER = NUM_RBF * EDGE_FEAT_DIM

TAB = 256
BN = 1000
BE = 2000

NC = 2
NS = 16
NW = NC * NS
EPW = E // NW
CCH = 400
NCH = EPW // CCH
WCH = 104
NWB = N // WCH
WTAIL = N - NWB * WCH

EPT = E // NS
SCCH = 200

_f32 = jnp.float32


@functools.lru_cache(maxsize=1)
def _sc_mesh():
    return plsc.VectorSubcoreMesh(
        core_axis_name="c", subcore_axis_name="s",
        num_cores=NC, num_subcores=NS)


def _ln(v, g, b):
    mu = jnp.mean(v, axis=1, keepdims=True)
    var = jnp.mean((v - mu) ** 2, axis=1, keepdims=True)
    return (v - mu) * jax.lax.rsqrt(var + 1e-5) * g + b


def _dot(a, b):
    return jnp.dot(a, b, preferred_element_type=_f32)


def _node_pre(h_ref, x_ref, qwT_ref, qb_ref, w1hT_ref, tabA_ref, tabB_ref):
    h = h_ref[...]
    hk = _dot(h, w1hT_ref[...])
    q = _dot(h, qwT_ref[...]) + qb_ref[...]
    xpad = jnp.concatenate(
        [x_ref[...], jnp.zeros((h.shape[0], TAB - HIDDEN - 3), _f32)], axis=1)
    tabA_ref[...] = jnp.concatenate([hk, xpad], axis=1)
    tabB_ref[...] = jnp.concatenate([q, xpad], axis=1)


def _run_node_pre(h, x, qwT, qb, w1hT):
    grid = (N // BN,)
    full = lambda shp: pl.BlockSpec(shp, lambda i: (0, 0))
    return pl.pallas_call(
        _node_pre,
        grid=grid,
        in_specs=[
            pl.BlockSpec((BN, HIDDEN), lambda i: (i, 0)),
            pl.BlockSpec((BN, 3), lambda i: (i, 0)),
            full((HIDDEN, HIDDEN)),
            full((1, HIDDEN)),
            full((HIDDEN, HIDDEN)),
        ],
        out_specs=[
            pl.BlockSpec((BN, TAB), lambda i: (i, 0)),
            pl.BlockSpec((BN, TAB), lambda i: (i, 0)),
        ],
        out_shape=[
            jax.ShapeDtypeStruct((N, TAB), _f32),
            jax.ShapeDtypeStruct((N, TAB), _f32),
        ],
    )(h, x, qwT, qb, w1hT)


def _edge_mlp(gA_ref, gB_ref, ea_ref, w1eT_ref, kvb1_ref, lng_ref, lnb_ref,
              w2T_ref, kvb2_ref, abw1T_ref, abb1_ref, ablng_ref, ablnb_ref,
              abw2T_ref, abb2_ref, combo_ref, v_ref):
    a = gA_ref[...]
    b = gB_ref[...]
    xs = a[:, HIDDEN:HIDDEN + 3]
    xd = b[:, HIDDEN:HIDDEN + 3]
    rel = xd - xs
    d2 = jnp.sum(rel * rel, axis=1, keepdims=True)
    d = jnp.sqrt(d2 + 1e-8)
    delta = CUTOFF / (NUM_RBF - 1)
    off = lax.broadcasted_iota(jnp.int32, (1, NUM_RBF), 1).astype(_f32) * delta
    df = jnp.exp((-0.5 / (delta * delta)) * (d - off) ** 2)
    ci = lax.broadcasted_iota(jnp.int32, (NUM_RBF, OUTER), 1)
    ri = lax.broadcasted_iota(jnp.int32, (NUM_RBF, OUTER), 0)
    Rm = (ci // EDGE_FEAT_DIM == ri).astype(_f32)
    cj = lax.broadcasted_iota(jnp.int32, (EDGE_FEAT_DIM, OUTER), 1)
    rj = lax.broadcasted_iota(jnp.int32, (EDGE_FEAT_DIM, OUTER), 0)
    Sm = (cj % EDGE_FEAT_DIM == rj).astype(_f32)
    ef = _dot(df, Rm) * _dot(ea_ref[...], Sm)
    t1 = a[:, :HIDDEN] + _dot(ef, w1eT_ref[...]) + kvb1_ref[...]
    t1 = jnp.maximum(_ln(t1, lng_ref[...], lnb_ref[...]), 0.0)
    kv = _dot(t1, w2T_ref[...]) + kvb2_ref[...]
    k = kv[:, :HIDDEN]
    v = kv[:, HIDDEN:]
    q = b[:, :HIDDEN]
    di = lax.broadcasted_iota(jnp.int32, (HIDDEN, N_HEADS), 0)
    hi = lax.broadcasted_iota(jnp.int32, (HIDDEN, N_HEADS), 1)
    Hm = (di // HEAD_DIM == hi).astype(_f32)
    logits = _dot(q * k, Hm) * SCALE
    tb = _dot(ef, abw1T_ref[...]) + abb1_ref[...]
    tb = jnp.maximum(_ln(tb, ablng_ref[...], ablnb_ref[...]), 0.0)
    ab = _dot(tb, abw2T_ref[...]) + abb2_ref[...]
    n = rel.shape[0]
    combo_ref[...] = jnp.concatenate(
        [jnp.exp(logits + ab), rel / (d + 1.0),
         jnp.zeros((n, HIDDEN - N_HEADS - 3), _f32)], axis=1)
    v_ref[...] = v


def _run_edge_mlp(gA, gB, ea, w1eT, kvb1, lng, lnb, w2T, kvb2,
                  abw1T, abb1, ablng, ablnb, abw2T, abb2):
    grid = (E // BE,)
    full = lambda shp: pl.BlockSpec(shp, lambda i: (0, 0))
    return pl.pallas_call(
        _edge_mlp,
        grid=grid,
        in_specs=[
            pl.BlockSpec((BE, TAB), lambda i: (i, 0)),
            pl.BlockSpec((BE, TAB), lambda i: (i, 0)),
            pl.BlockSpec((BE, EDGE_FEAT_DIM), lambda i: (i, 0)),
            full((OUTER, HIDDEN)),
            full((1, HIDDEN)),
            full((1, HIDDEN)),
            full((1, HIDDEN)),
            full((HIDDEN, 2 * HIDDEN)),
            full((1, 2 * HIDDEN)),
            full((OUTER, HIDDEN)),
            full((1, HIDDEN)),
            full((1, HIDDEN)),
            full((1, HIDDEN)),
            full((HIDDEN, N_HEADS)),
            full((1, N_HEADS)),
        ],
        out_specs=[
            pl.BlockSpec((BE, HIDDEN), lambda i: (i, 0)),
            pl.BlockSpec((BE, HIDDEN), lambda i: (i, 0)),
        ],
        out_shape=[
            jax.ShapeDtypeStruct((E, HIDDEN), _f32),
            jax.ShapeDtypeStruct((E, HIDDEN), _f32),
        ],
    )(gA, gB, ea, w1eT, kvb1, lng, lnb, w2T, kvb2,
      abw1T, abb1, ablng, ablnb, abw2T, abb2)


def _sum2(pa_ref, pb_ref, out_ref):
    out_ref[...] = pa_ref[0] + pb_ref[0]


def _run_sum2(parts):
    grid = (N // BN,)
    return pl.pallas_call(
        _sum2,
        grid=grid,
        in_specs=[
            pl.BlockSpec((1, BN, HIDDEN), lambda i: (0, i, 0)),
            pl.BlockSpec((1, BN, HIDDEN), lambda i: (1, i, 0)),
        ],
        out_specs=pl.BlockSpec((BN, HIDDEN), lambda i: (i, 0)),
        out_shape=jax.ShapeDtypeStruct((N, HIDDEN), _f32),
    )(parts, parts)


def _edge2(combo_ref, sg_ref, v_ref, xw1T_ref, xb1_ref, xw2_ref,
           msg_ref, vecp_ref):
    combo = combo_ref[...]
    w = combo[:, :N_HEADS] / (sg_ref[...][:, :N_HEADS] + 1e-16)
    relod = combo[:, N_HEADS:N_HEADS + 3]
    hi = lax.broadcasted_iota(jnp.int32, (N_HEADS, HIDDEN), 0)
    di = lax.broadcasted_iota(jnp.int32, (N_HEADS, HIDDEN), 1)
    Bm = (di // HEAD_DIM == hi).astype(_f32)
    msg = _dot(w, Bm) * v_ref[...]
    m1 = _dot(msg, xw1T_ref[...]) + xb1_ref[...]
    m1 = m1 * jax.nn.sigmoid(m1)
    cc = jnp.sum(m1 * xw2_ref[...], axis=1, keepdims=True)
    coef = jnp.tanh(cc)
    vec = relod * coef
    msg_ref[...] = msg
    vecp_ref[...] = jnp.concatenate(
        [vec, jnp.zeros((vec.shape[0], HIDDEN - 3), _f32)], axis=1)


def _run_edge2(combo, sg, v, xw1T, xb1, xw2):
    grid = (E // BE,)
    full = lambda shp: pl.BlockSpec(shp, lambda i: (0, 0))
    return pl.pallas_call(
        _edge2,
        grid=grid,
        in_specs=[
            pl.BlockSpec((BE, HIDDEN), lambda i: (i, 0)),
            pl.BlockSpec((BE, HIDDEN), lambda i: (i, 0)),
            pl.BlockSpec((BE, HIDDEN), lambda i: (i, 0)),
            full((HIDDEN, HIDDEN)),
            full((1, HIDDEN)),
            full((1, HIDDEN)),
        ],
        out_specs=[
            pl.BlockSpec((BE, HIDDEN), lambda i: (i, 0)),
            pl.BlockSpec((BE, HIDDEN), lambda i: (i, 0)),
        ],
        out_shape=[
            jax.ShapeDtypeStruct((E, HIDDEN), _f32),
            jax.ShapeDtypeStruct((E, HIDDEN), _f32),
        ],
    )(combo, sg, v, xw1T, xb1, xw2)


def _node_post(pa_ref, va_ref, h_ref, x_ref, mask_ref,
               outwT_ref, outb_ref, nmw1T_ref, nmb1_ref, nmlng_ref,
               nmlnb_ref, nmw2T_ref, nmb2_ref, hnew_ref, xnew_ref):
    agg = pa_ref[0]
    dxv = va_ref[0][:, :3]
    h = h_ref[...]
    aggo = _dot(agg, outwT_ref[...]) + outb_ref[...]
    tn = jnp.concatenate([aggo, h], axis=1)
    tn = _dot(tn, nmw1T_ref[...]) + nmb1_ref[...]
    tn = jnp.maximum(_ln(tn, nmlng_ref[...], nmlnb_ref[...]), 0.0)
    tn = _dot(tn, nmw2T_ref[...]) + nmb2_ref[...]
    hnew_ref[...] = h + tn
    xnew_ref[...] = x_ref[...] + dxv * mask_ref[...]


def _run_node_post(parts, h, x, maskf, outwT, outb, nmw1T, nmb1,
                   nmlng, nmlnb, nmw2T, nmb2):
    grid = (N // BN,)
    full = lambda shp: pl.BlockSpec(shp, lambda i: (0, 0))
    return pl.pallas_call(
        _node_post,
        grid=grid,
        in_specs=[
            pl.BlockSpec((1, BN, HIDDEN), lambda i: (0, i, 0)),
            pl.BlockSpec((1, BN, HIDDEN), lambda i: (1, i, 0)),
            pl.BlockSpec((BN, HIDDEN), lambda i: (i, 0)),
            pl.BlockSpec((BN, 3), lambda i: (i, 0)),
            pl.BlockSpec((BN, 1), lambda i: (i, 0)),
            full((HIDDEN, HIDDEN)),
            full((1, HIDDEN)),
            full((2 * HIDDEN, HIDDEN)),
            full((1, HIDDEN)),
            full((1, HIDDEN)),
            full((1, HIDDEN)),
            full((HIDDEN, HIDDEN)),
            full((1, HIDDEN)),
        ],
        out_specs=[
            pl.BlockSpec((BN, HIDDEN), lambda i: (i, 0)),
            pl.BlockSpec((BN, 3), lambda i: (i, 0)),
        ],
        out_shape=[
            jax.ShapeDtypeStruct((N, HIDDEN), _f32),
            jax.ShapeDtypeStruct((N, 3), _f32),
        ],
    )(parts, parts, h, x, maskf, outwT, outb, nmw1T, nmb1,
      nmlng, nmlnb, nmw2T, nmb2)


def _sc_gather2_body(tabA, tabB, src, dst, gA, gB, idxv, bufv, sem):
    wid = lax.axis_index("s") * NC + lax.axis_index("c")
    base0 = wid * EPW

    def chunk(c, carry):
        base = base0 + c * CCH
        pltpu.sync_copy(src.at[pl.ds(base, CCH)], idxv)
        pltpu.async_copy(tabA.at[idxv], bufv, sem).wait()
        pltpu.sync_copy(bufv, gA.at[pl.ds(base, CCH), :])
        pltpu.sync_copy(dst.at[pl.ds(base, CCH)], idxv)
        pltpu.async_copy(tabB.at[idxv], bufv, sem).wait()
        pltpu.sync_copy(bufv, gB.at[pl.ds(base, CCH), :])
        return carry

    lax.fori_loop(0, NCH, chunk, 0)


def _run_sc_gather2(tabA, tabB, src, dst):
    return pl.kernel(
        _sc_gather2_body,
        out_type=[
            jax.ShapeDtypeStruct((E, TAB), _f32),
            jax.ShapeDtypeStruct((E, TAB), _f32),
        ],
        mesh=_sc_mesh(),
        scratch_types=[
            pltpu.VMEM((CCH,), jnp.int32),
            pltpu.VMEM((CCH, TAB), _f32),
            pltpu.SemaphoreType.DMA,
        ],
    )(tabA, tabB, src, dst)


def _acc_zero(zrows, acc_ref, sid):
    for w in range(NWB // NS):
        r = (sid * (NWB // NS) + w) * WCH
        pltpu.sync_copy(zrows.at[pl.ds(r, WCH), :],
                        acc_ref.at[pl.ds(r, WCH), :])

    @pl.when(sid == 0)
    def _():
        pltpu.sync_copy(zrows.at[pl.ds(NWB * WCH, WTAIL), :],
                        acc_ref.at[pl.ds(NWB * WCH, WTAIL), :])


def _acc_writeback(acc_ref, out, cid, sid, wbuf, tbuf):
    for w in range(NWB // NS):
        r = (sid * (NWB // NS) + w) * WCH
        pltpu.sync_copy(acc_ref.at[pl.ds(r, WCH), :], wbuf)
        pltpu.sync_copy(wbuf, out.at[cid, pl.ds(r, WCH), :])

    @pl.when(sid == 0)
    def _():
        pltpu.sync_copy(acc_ref.at[pl.ds(NWB * WCH, WTAIL), :], tbuf)
        pltpu.sync_copy(tbuf, out.at[cid, pl.ds(NWB * WCH, WTAIL), :])


def _scatter_loop(vals, dst, acc_ref, idxv, bufv, base0, nchunks):
    def chunk(c, carry):
        base = base0 + c * SCCH
        pltpu.sync_copy(dst.at[pl.ds(base, SCCH)], idxv)
        pltpu.sync_copy(vals.at[pl.ds(base, SCCH), :], bufv)
        pltpu.sync_copy(bufv, acc_ref.at[idxv], add=True)
        return carry

    lax.fori_loop(0, nchunks, chunk, 0)


def _sc_scatter_part_body(vals, dst, zrows, out, idxv, bufv, wbuf, tbuf,
                          acc_ref):
    cid = lax.axis_index("c")
    sid = lax.axis_index("s")
    _acc_zero(zrows, acc_ref, sid)
    plsc.subcore_barrier()
    base0 = (cid * NS + sid) * EPW
    _scatter_loop(vals, dst, acc_ref, idxv, bufv, base0, EPW // SCCH)
    plsc.subcore_barrier()
    _acc_writeback(acc_ref, out, cid, sid, wbuf, tbuf)


def _run_sc_scatter_part(vals, dst, zrows):
    return pl.kernel(
        _sc_scatter_part_body,
        out_type=jax.ShapeDtypeStruct((NC, N, HIDDEN), _f32),
        mesh=_sc_mesh(),
        scratch_types=[
            pltpu.VMEM((SCCH,), jnp.int32),
            pltpu.VMEM((SCCH, HIDDEN), _f32),
            pltpu.VMEM((WCH, HIDDEN), _f32),
            pltpu.VMEM((WTAIL, HIDDEN), _f32),
            pltpu.VMEM_SHARED((N, HIDDEN), _f32),
        ],
    )(vals, dst, zrows)


def _sc_scatter_dual_body(valsA, valsB, dst, zrows, out, idxv, bufv, wbuf,
                          tbuf, acc_ref):
    cid = lax.axis_index("c")
    sid = lax.axis_index("s")
    _acc_zero(zrows, acc_ref, sid)
    plsc.subcore_barrier()
    base0 = sid * EPT

    @pl.when(cid == 0)
    def _():
        _scatter_loop(valsA, dst, acc_ref, idxv, bufv, base0, EPT // SCCH)

    @pl.when(cid == 1)
    def _():
        _scatter_loop(valsB, dst, acc_ref, idxv, bufv, base0, EPT // SCCH)

    plsc.subcore_barrier()
    _acc_writeback(acc_ref, out, cid, sid, wbuf, tbuf)


def _run_sc_scatter_dual(valsA, valsB, dst, zrows):
    return pl.kernel(
        _sc_scatter_dual_body,
        out_type=jax.ShapeDtypeStruct((NC, N, HIDDEN), _f32),
        mesh=_sc_mesh(),
        scratch_types=[
            pltpu.VMEM((SCCH,), jnp.int32),
            pltpu.VMEM((SCCH, HIDDEN), _f32),
            pltpu.VMEM((WCH, HIDDEN), _f32),
            pltpu.VMEM((WTAIL, HIDDEN), _f32),
            pltpu.VMEM_SHARED((N, HIDDEN), _f32),
        ],
    )(valsA, valsB, dst, zrows)


def _sc_gather1_body(tab, dst, out, idxv, bufv, sem):
    wid = lax.axis_index("s") * NC + lax.axis_index("c")
    base0 = wid * EPW

    def chunk(c, carry):
        base = base0 + c * CCH
        pltpu.sync_copy(dst.at[pl.ds(base, CCH)], idxv)
        pltpu.async_copy(tab.at[idxv], bufv, sem).wait()
        pltpu.sync_copy(bufv, out.at[pl.ds(base, CCH), :])
        return carry

    lax.fori_loop(0, NCH, chunk, 0)


def _run_sc_gather1(tab, dst):
    return pl.kernel(
        _sc_gather1_body,
        out_type=jax.ShapeDtypeStruct((E, HIDDEN), _f32),
        mesh=_sc_mesh(),
        scratch_types=[
            pltpu.VMEM((CCH,), jnp.int32),
            pltpu.VMEM((CCH, HIDDEN), _f32),
            pltpu.SemaphoreType.DMA,
        ],
    )(tab, dst)


def kernel(h, x, edge_index, mask_ligand, edge_attr, params):
    p = params
    row = lambda v: v.reshape(1, -1)
    qwT = p["q_w"].T
    w1hT = p["kv_w1"][:, :HIDDEN].T
    w1eT = p["kv_w1"][:, HIDDEN:].T
    w2T = p["kv_w2"].T
    abw1T = p["ab_w1"].T
    abw2T = p["ab_w2"].T
    outwT = p["out_w"].T
    nmw1T = p["nm_w1"].T
    nmw2T = p["nm_w2"].T
    xw1T = p["x_w1"].T
    maskf = mask_ligand.astype(_f32).reshape(N, 1)
    src = edge_index[0]
    dst = edge_index[1]

    tabA, tabB = _run_node_pre(h, x, qwT, row(p["q_b"]), w1hT)

    zrows = jnp.zeros((N, HIDDEN), _f32)
    gA, gB = _run_sc_gather2(tabA, tabB, src, dst)

    combo, v = _run_edge_mlp(
        gA, gB, edge_attr, w1eT, row(p["kv_b1"]), row(p["kv_ln_g"]),
        row(p["kv_ln_b"]), w2T, row(p["kv_b2"]), abw1T, row(p["ab_b1"]),
        row(p["ab_ln_g"]), row(p["ab_ln_b"]), abw2T, row(p["ab_b2"]))

    s_parts = _run_sc_scatter_part(combo, dst, zrows)
    s = _run_sum2(s_parts)
    sg = _run_sc_gather1(s, dst)

    msg, vecp = _run_edge2(combo, sg, v, xw1T, row(p["x_b1"]),
                           p["x_w2"].reshape(1, HIDDEN))

    parts = _run_sc_scatter_dual(msg, vecp, dst, zrows)

    h_new, x_new = _run_node_post(
        parts, h, x, maskf, outwT, row(p["out_b"]), nmw1T, row(p["nm_b1"]),
        row(p["nm_ln_g"]), row(p["nm_ln_b"]), nmw2T, row(p["nm_b2"]))
    return (h_new, x_new)


# speedup vs baseline: 3.9627x; 1.0232x over previous
import functools
import math

import jax
import jax.numpy as jnp
from jax import lax
from jax.experimental import pallas as pl
from jax.experimental.pallas import tpu as pltpu
from jax.experimental.pallas import tpu_sc as plsc

N = 10000
E = 320000
HIDDEN = 128
EDGE_FEAT_DIM = 4
NUM_RBF = 16
N_HEADS = 16
HEAD_DIM = HIDDEN // N_HEADS
CUTOFF = 10.0
SCALE = 1.0 / math.sqrt(HEAD_DIM)
OUTER = NUM_RBF * EDGE_FEAT_DIM

TAB = 256
BN = 1000
BE = 2000

NC = 2
NS = 16
NW = NC * NS
EPW = E // NW
CCH = 200
NCH = EPW // CCH
WCH = 104
NWB = N // WCH
WTAIL = N - NWB * WCH

EPT = E // NS
SCCH = 200

_f32 = jnp.float32


@functools.lru_cache(maxsize=1)
def _sc_mesh():
    return plsc.VectorSubcoreMesh(
        core_axis_name="c", subcore_axis_name="s",
        num_cores=NC, num_subcores=NS)


def _ln(v, g, b):
    mu = jnp.mean(v, axis=1, keepdims=True)
    var = jnp.mean((v - mu) ** 2, axis=1, keepdims=True)
    return (v - mu) * jax.lax.rsqrt(var + 1e-5) * g + b


def _dot(a, b):
    return jnp.dot(a, b, preferred_element_type=_f32)


def _node_pre(h_ref, x_ref, qwT_ref, qb_ref, w1hT_ref, tabA_ref, tabB_ref):
    h = h_ref[...]
    hk = _dot(h, w1hT_ref[...])
    q = _dot(h, qwT_ref[...]) + qb_ref[...]
    xpad = jnp.concatenate(
        [x_ref[...], jnp.zeros((h.shape[0], TAB - HIDDEN - 3), _f32)], axis=1)
    tabA_ref[...] = jnp.concatenate([hk, xpad], axis=1)
    tabB_ref[...] = jnp.concatenate([q, xpad], axis=1)


def _run_node_pre(h, x, qwT, qb, w1hT):
    grid = (N // BN,)
    full = lambda shp: pl.BlockSpec(shp, lambda i: (0, 0))
    return pl.pallas_call(
        _node_pre,
        grid=grid,
        in_specs=[
            pl.BlockSpec((BN, HIDDEN), lambda i: (i, 0)),
            pl.BlockSpec((BN, 3), lambda i: (i, 0)),
            full((HIDDEN, HIDDEN)),
            full((1, HIDDEN)),
            full((HIDDEN, HIDDEN)),
        ],
        out_specs=[
            pl.BlockSpec((BN, TAB), lambda i: (i, 0)),
            pl.BlockSpec((BN, TAB), lambda i: (i, 0)),
        ],
        out_shape=[
            jax.ShapeDtypeStruct((N, TAB), _f32),
            jax.ShapeDtypeStruct((N, TAB), _f32),
        ],
    )(h, x, qwT, qb, w1hT)


def _edge_mlp(gA_ref, gB_ref, ea_ref, w1eT_ref, kvb1_ref, lng_ref, lnb_ref,
              w2T_ref, kvb2_ref, abw1T_ref, abb1_ref, ablng_ref, ablnb_ref,
              abw2T_ref, abb2_ref, combo_ref, v_ref):
    a = gA_ref[...]
    b = gB_ref[...]
    xs = a[:, HIDDEN:HIDDEN + 3]
    xd = b[:, HIDDEN:HIDDEN + 3]
    rel = xd - xs
    d2 = jnp.sum(rel * rel, axis=1, keepdims=True)
    d = jnp.sqrt(d2 + 1e-8)
    delta = CUTOFF / (NUM_RBF - 1)
    off = lax.broadcasted_iota(jnp.int32, (1, NUM_RBF), 1).astype(_f32) * delta
    df = jnp.exp((-0.5 / (delta * delta)) * (d - off) ** 2)
    ci = lax.broadcasted_iota(jnp.int32, (NUM_RBF, OUTER), 1)
    ri = lax.broadcasted_iota(jnp.int32, (NUM_RBF, OUTER), 0)
    Rm = (ci // EDGE_FEAT_DIM == ri).astype(_f32)
    cj = lax.broadcasted_iota(jnp.int32, (EDGE_FEAT_DIM, OUTER), 1)
    rj = lax.broadcasted_iota(jnp.int32, (EDGE_FEAT_DIM, OUTER), 0)
    Sm = (cj % EDGE_FEAT_DIM == rj).astype(_f32)
    ef = _dot(df, Rm) * _dot(ea_ref[...], Sm)
    t1 = a[:, :HIDDEN] + _dot(ef, w1eT_ref[...]) + kvb1_ref[...]
    t1 = jnp.maximum(_ln(t1, lng_ref[...], lnb_ref[...]), 0.0)
    kv = _dot(t1, w2T_ref[...]) + kvb2_ref[...]
    k = kv[:, :HIDDEN]
    v = kv[:, HIDDEN:]
    q = b[:, :HIDDEN]
    di = lax.broadcasted_iota(jnp.int32, (HIDDEN, N_HEADS), 0)
    hi = lax.broadcasted_iota(jnp.int32, (HIDDEN, N_HEADS), 1)
    Hm = (di // HEAD_DIM == hi).astype(_f32)
    logits = _dot(q * k, Hm) * SCALE
    tb = _dot(ef, abw1T_ref[...]) + abb1_ref[...]
    tb = jnp.maximum(_ln(tb, ablng_ref[...], ablnb_ref[...]), 0.0)
    ab = _dot(tb, abw2T_ref[...]) + abb2_ref[...]
    n = rel.shape[0]
    combo_ref[...] = jnp.concatenate(
        [jnp.exp(logits + ab), rel / (d + 1.0),
         jnp.zeros((n, HIDDEN - N_HEADS - 3), _f32)], axis=1)
    v_ref[...] = v


def _run_edge_mlp(gA, gB, ea, w1eT, kvb1, lng, lnb, w2T, kvb2,
                  abw1T, abb1, ablng, ablnb, abw2T, abb2):
    grid = (E // BE,)
    full = lambda shp: pl.BlockSpec(shp, lambda i: (0, 0))
    return pl.pallas_call(
        _edge_mlp,
        grid=grid,
        in_specs=[
            pl.BlockSpec((BE, TAB), lambda i: (i, 0)),
            pl.BlockSpec((BE, TAB), lambda i: (i, 0)),
            pl.BlockSpec((BE, EDGE_FEAT_DIM), lambda i: (i, 0)),
            full((OUTER, HIDDEN)),
            full((1, HIDDEN)),
            full((1, HIDDEN)),
            full((1, HIDDEN)),
            full((HIDDEN, 2 * HIDDEN)),
            full((1, 2 * HIDDEN)),
            full((OUTER, HIDDEN)),
            full((1, HIDDEN)),
            full((1, HIDDEN)),
            full((1, HIDDEN)),
            full((HIDDEN, N_HEADS)),
            full((1, N_HEADS)),
        ],
        out_specs=[
            pl.BlockSpec((BE, HIDDEN), lambda i: (i, 0)),
            pl.BlockSpec((BE, HIDDEN), lambda i: (i, 0)),
        ],
        out_shape=[
            jax.ShapeDtypeStruct((E, HIDDEN), _f32),
            jax.ShapeDtypeStruct((E, HIDDEN), _f32),
        ],
    )(gA, gB, ea, w1eT, kvb1, lng, lnb, w2T, kvb2,
      abw1T, abb1, ablng, ablnb, abw2T, abb2)


def _sum2(pa_ref, pb_ref, out_ref):
    out_ref[...] = pa_ref[0] + pb_ref[0]


def _run_sum2(parts):
    grid = (N // BN,)
    return pl.pallas_call(
        _sum2,
        grid=grid,
        in_specs=[
            pl.BlockSpec((1, BN, HIDDEN), lambda i: (0, i, 0)),
            pl.BlockSpec((1, BN, HIDDEN), lambda i: (1, i, 0)),
        ],
        out_specs=pl.BlockSpec((BN, HIDDEN), lambda i: (i, 0)),
        out_shape=jax.ShapeDtypeStruct((N, HIDDEN), _f32),
    )(parts, parts)


def _edge2(combo_ref, sg_ref, v_ref, xw1T_ref, xb1_ref, xw2_ref,
           msg_ref, vecp_ref):
    combo = combo_ref[...]
    w = combo[:, :N_HEADS] / (sg_ref[...][:, :N_HEADS] + 1e-16)
    relod = combo[:, N_HEADS:N_HEADS + 3]
    hi = lax.broadcasted_iota(jnp.int32, (N_HEADS, HIDDEN), 0)
    di = lax.broadcasted_iota(jnp.int32, (N_HEADS, HIDDEN), 1)
    Bm = (di // HEAD_DIM == hi).astype(_f32)
    msg = _dot(w, Bm) * v_ref[...]
    m1 = _dot(msg, xw1T_ref[...]) + xb1_ref[...]
    m1 = m1 * jax.nn.sigmoid(m1)
    cc = jnp.sum(m1 * xw2_ref[...], axis=1, keepdims=True)
    coef = jnp.tanh(cc)
    vec = relod * coef
    msg_ref[...] = msg
    vecp_ref[...] = jnp.concatenate(
        [vec, jnp.zeros((vec.shape[0], HIDDEN - 3), _f32)], axis=1)


def _run_edge2(combo, sg, v, xw1T, xb1, xw2):
    grid = (E // BE,)
    full = lambda shp: pl.BlockSpec(shp, lambda i: (0, 0))
    return pl.pallas_call(
        _edge2,
        grid=grid,
        in_specs=[
            pl.BlockSpec((BE, HIDDEN), lambda i: (i, 0)),
            pl.BlockSpec((BE, HIDDEN), lambda i: (i, 0)),
            pl.BlockSpec((BE, HIDDEN), lambda i: (i, 0)),
            full((HIDDEN, HIDDEN)),
            full((1, HIDDEN)),
            full((1, HIDDEN)),
        ],
        out_specs=[
            pl.BlockSpec((BE, HIDDEN), lambda i: (i, 0)),
            pl.BlockSpec((BE, HIDDEN), lambda i: (i, 0)),
        ],
        out_shape=[
            jax.ShapeDtypeStruct((E, HIDDEN), _f32),
            jax.ShapeDtypeStruct((E, HIDDEN), _f32),
        ],
    )(combo, sg, v, xw1T, xb1, xw2)


def _node_post(pa_ref, va_ref, h_ref, x_ref, mask_ref,
               outwT_ref, outb_ref, nmw1T_ref, nmb1_ref, nmlng_ref,
               nmlnb_ref, nmw2T_ref, nmb2_ref, hnew_ref, xnew_ref):
    agg = pa_ref[0]
    dxv = va_ref[0][:, :3]
    h = h_ref[...]
    aggo = _dot(agg, outwT_ref[...]) + outb_ref[...]
    tn = jnp.concatenate([aggo, h], axis=1)
    tn = _dot(tn, nmw1T_ref[...]) + nmb1_ref[...]
    tn = jnp.maximum(_ln(tn, nmlng_ref[...], nmlnb_ref[...]), 0.0)
    tn = _dot(tn, nmw2T_ref[...]) + nmb2_ref[...]
    hnew_ref[...] = h + tn
    xnew_ref[...] = x_ref[...] + dxv * mask_ref[...]


def _run_node_post(parts, h, x, maskf, outwT, outb, nmw1T, nmb1,
                   nmlng, nmlnb, nmw2T, nmb2):
    grid = (N // BN,)
    full = lambda shp: pl.BlockSpec(shp, lambda i: (0, 0))
    return pl.pallas_call(
        _node_post,
        grid=grid,
        in_specs=[
            pl.BlockSpec((1, BN, HIDDEN), lambda i: (0, i, 0)),
            pl.BlockSpec((1, BN, HIDDEN), lambda i: (1, i, 0)),
            pl.BlockSpec((BN, HIDDEN), lambda i: (i, 0)),
            pl.BlockSpec((BN, 3), lambda i: (i, 0)),
            pl.BlockSpec((BN, 1), lambda i: (i, 0)),
            full((HIDDEN, HIDDEN)),
            full((1, HIDDEN)),
            full((2 * HIDDEN, HIDDEN)),
            full((1, HIDDEN)),
            full((1, HIDDEN)),
            full((1, HIDDEN)),
            full((HIDDEN, HIDDEN)),
            full((1, HIDDEN)),
        ],
        out_specs=[
            pl.BlockSpec((BN, HIDDEN), lambda i: (i, 0)),
            pl.BlockSpec((BN, 3), lambda i: (i, 0)),
        ],
        out_shape=[
            jax.ShapeDtypeStruct((N, HIDDEN), _f32),
            jax.ShapeDtypeStruct((N, 3), _f32),
        ],
    )(parts, parts, h, x, maskf, outwT, outb, nmw1T, nmb1,
      nmlng, nmlnb, nmw2T, nmb2)


def _sc_gather2_body(tabA, tabB, src, dst, gA, gB,
                     idxa, idxb, bufa, bufb, sema, semb):
    wid = lax.axis_index("s") * NC + lax.axis_index("c")
    base0 = wid * EPW

    def chunk(c, carry):
        base = base0 + c * CCH
        pltpu.sync_copy(src.at[pl.ds(base, CCH)], idxa)
        cpa = pltpu.async_copy(tabA.at[idxa], bufa, sema)
        pltpu.sync_copy(dst.at[pl.ds(base, CCH)], idxb)
        cpb = pltpu.async_copy(tabB.at[idxb], bufb, semb)
        cpa.wait()
        pltpu.sync_copy(bufa, gA.at[pl.ds(base, CCH), :])
        cpb.wait()
        pltpu.sync_copy(bufb, gB.at[pl.ds(base, CCH), :])
        return carry

    lax.fori_loop(0, NCH, chunk, 0)


def _run_sc_gather2(tabA, tabB, src, dst):
    return pl.kernel(
        _sc_gather2_body,
        out_type=[
            jax.ShapeDtypeStruct((E, TAB), _f32),
            jax.ShapeDtypeStruct((E, TAB), _f32),
        ],
        mesh=_sc_mesh(),
        scratch_types=[
            pltpu.VMEM((CCH,), jnp.int32),
            pltpu.VMEM((CCH,), jnp.int32),
            pltpu.VMEM((CCH, TAB), _f32),
            pltpu.VMEM((CCH, TAB), _f32),
            pltpu.SemaphoreType.DMA,
            pltpu.SemaphoreType.DMA,
        ],
    )(tabA, tabB, src, dst)


def _acc_zero(zrows, acc_ref, sid):
    for w in range(NWB // NS):
        r = (sid * (NWB // NS) + w) * WCH
        pltpu.sync_copy(zrows.at[pl.ds(r, WCH), :],
                        acc_ref.at[pl.ds(r, WCH), :])

    @pl.when(sid == 0)
    def _():
        pltpu.sync_copy(zrows.at[pl.ds(NWB * WCH, WTAIL), :],
                        acc_ref.at[pl.ds(NWB * WCH, WTAIL), :])


def _acc_writeback(acc_ref, out, cid, sid, wbuf, tbuf):
    for w in range(NWB // NS):
        r = (sid * (NWB // NS) + w) * WCH
        pltpu.sync_copy(acc_ref.at[pl.ds(r, WCH), :], wbuf)
        pltpu.sync_copy(wbuf, out.at[cid, pl.ds(r, WCH), :])

    @pl.when(sid == 0)
    def _():
        pltpu.sync_copy(acc_ref.at[pl.ds(NWB * WCH, WTAIL), :], tbuf)
        pltpu.sync_copy(tbuf, out.at[cid, pl.ds(NWB * WCH, WTAIL), :])


def _scatter_loop(vals, dst, acc_ref, idxv, bufv, base0, nchunks):
    def chunk(c, carry):
        base = base0 + c * SCCH
        pltpu.sync_copy(dst.at[pl.ds(base, SCCH)], idxv)
        pltpu.sync_copy(vals.at[pl.ds(base, SCCH), :], bufv)
        pltpu.sync_copy(bufv, acc_ref.at[idxv], add=True)
        return carry

    lax.fori_loop(0, nchunks, chunk, 0)


def _sc_scatter_part_body(vals, dst, zrows, out, idxv, bufv, wbuf, tbuf,
                          acc_ref):
    cid = lax.axis_index("c")
    sid = lax.axis_index("s")
    _acc_zero(zrows, acc_ref, sid)
    plsc.subcore_barrier()
    base0 = (cid * NS + sid) * EPW
    _scatter_loop(vals, dst, acc_ref, idxv, bufv, base0, EPW // SCCH)
    plsc.subcore_barrier()
    _acc_writeback(acc_ref, out, cid, sid, wbuf, tbuf)


def _run_sc_scatter_part(vals, dst, zrows):
    return pl.kernel(
        _sc_scatter_part_body,
        out_type=jax.ShapeDtypeStruct((NC, N, HIDDEN), _f32),
        mesh=_sc_mesh(),
        scratch_types=[
            pltpu.VMEM((SCCH,), jnp.int32),
            pltpu.VMEM((SCCH, HIDDEN), _f32),
            pltpu.VMEM((WCH, HIDDEN), _f32),
            pltpu.VMEM((WTAIL, HIDDEN), _f32),
            pltpu.VMEM_SHARED((N, HIDDEN), _f32),
        ],
    )(vals, dst, zrows)


def _sc_scatter_dual_body(valsA, valsB, dst, zrows, out, idxv, bufv, wbuf,
                          tbuf, acc_ref):
    cid = lax.axis_index("c")
    sid = lax.axis_index("s")
    _acc_zero(zrows, acc_ref, sid)
    plsc.subcore_barrier()
    base0 = sid * EPT

    @pl.when(cid == 0)
    def _():
        _scatter_loop(valsA, dst, acc_ref, idxv, bufv, base0, EPT // SCCH)

    @pl.when(cid == 1)
    def _():
        _scatter_loop(valsB, dst, acc_ref, idxv, bufv, base0, EPT // SCCH)

    plsc.subcore_barrier()
    _acc_writeback(acc_ref, out, cid, sid, wbuf, tbuf)


def _run_sc_scatter_dual(valsA, valsB, dst, zrows):
    return pl.kernel(
        _sc_scatter_dual_body,
        out_type=jax.ShapeDtypeStruct((NC, N, HIDDEN), _f32),
        mesh=_sc_mesh(),
        scratch_types=[
            pltpu.VMEM((SCCH,), jnp.int32),
            pltpu.VMEM((SCCH, HIDDEN), _f32),
            pltpu.VMEM((WCH, HIDDEN), _f32),
            pltpu.VMEM((WTAIL, HIDDEN), _f32),
            pltpu.VMEM_SHARED((N, HIDDEN), _f32),
        ],
    )(valsA, valsB, dst, zrows)


def _sc_gather1_body(tab, dst, out, idxa, idxb, bufa, bufb, sema, semb):
    wid = lax.axis_index("s") * NC + lax.axis_index("c")
    base0 = wid * EPW

    def pair(c, carry):
        base = base0 + c * 2 * CCH
        pltpu.sync_copy(dst.at[pl.ds(base, CCH)], idxa)
        cpa = pltpu.async_copy(tab.at[idxa], bufa, sema)
        pltpu.sync_copy(dst.at[pl.ds(base + CCH, CCH)], idxb)
        cpb = pltpu.async_copy(tab.at[idxb], bufb, semb)
        cpa.wait()
        pltpu.sync_copy(bufa, out.at[pl.ds(base, CCH), :])
        cpb.wait()
        pltpu.sync_copy(bufb, out.at[pl.ds(base + CCH, CCH), :])
        return carry

    lax.fori_loop(0, NCH // 2, pair, 0)


def _run_sc_gather1(tab, dst):
    return pl.kernel(
        _sc_gather1_body,
        out_type=jax.ShapeDtypeStruct((E, HIDDEN), _f32),
        mesh=_sc_mesh(),
        scratch_types=[
            pltpu.VMEM((CCH,), jnp.int32),
            pltpu.VMEM((CCH,), jnp.int32),
            pltpu.VMEM((CCH, HIDDEN), _f32),
            pltpu.VMEM((CCH, HIDDEN), _f32),
            pltpu.SemaphoreType.DMA,
            pltpu.SemaphoreType.DMA,
        ],
    )(tab, dst)


def kernel(h, x, edge_index, mask_ligand, edge_attr, params):
    p = params
    row = lambda v: v.reshape(1, -1)
    qwT = p["q_w"].T
    w1hT = p["kv_w1"][:, :HIDDEN].T
    w1eT = p["kv_w1"][:, HIDDEN:].T
    w2T = p["kv_w2"].T
    abw1T = p["ab_w1"].T
    abw2T = p["ab_w2"].T
    outwT = p["out_w"].T
    nmw1T = p["nm_w1"].T
    nmw2T = p["nm_w2"].T
    xw1T = p["x_w1"].T
    maskf = mask_ligand.astype(_f32).reshape(N, 1)
    src = edge_index[0]
    dst = edge_index[1]

    tabA, tabB = _run_node_pre(h, x, qwT, row(p["q_b"]), w1hT)

    zrows = jnp.zeros((N, HIDDEN), _f32)
    gA, gB = _run_sc_gather2(tabA, tabB, src, dst)

    combo, v = _run_edge_mlp(
        gA, gB, edge_attr, w1eT, row(p["kv_b1"]), row(p["kv_ln_g"]),
        row(p["kv_ln_b"]), w2T, row(p["kv_b2"]), abw1T, row(p["ab_b1"]),
        row(p["ab_ln_g"]), row(p["ab_ln_b"]), abw2T, row(p["ab_b2"]))

    s_parts = _run_sc_scatter_part(combo, dst, zrows)
    s = _run_sum2(s_parts)
    sg = _run_sc_gather1(s, dst)

    msg, vecp = _run_edge2(combo, sg, v, xw1T, row(p["x_b1"]),
                           p["x_w2"].reshape(1, HIDDEN))

    parts = _run_sc_scatter_dual(msg, vecp, dst, zrows)

    h_new, x_new = _run_node_post(
        parts, h, x, maskf, outwT, row(p["out_b"]), nmw1T, row(p["nm_b1"]),
        row(p["nm_ln_g"]), row(p["nm_ln_b"]), nmw2T, row(p["nm_b2"]))
    return (h_new, x_new)


# speedup vs baseline: 4.0994x; 1.0345x over previous
import functools
import math

import jax
import jax.numpy as jnp
from jax import lax
from jax.experimental import pallas as pl
from jax.experimental.pallas import tpu as pltpu
from jax.experimental.pallas import tpu_sc as plsc

N = 10000
E = 320000
HIDDEN = 128
EDGE_FEAT_DIM = 4
NUM_RBF = 16
N_HEADS = 16
HEAD_DIM = HIDDEN // N_HEADS
CUTOFF = 10.0
SCALE = 1.0 / math.sqrt(HEAD_DIM)
OUTER = NUM_RBF * EDGE_FEAT_DIM

TAB = 256
BN = 1000
BE = 2000

NC = 2
NS = 16
NW = NC * NS
EPW = E // NW
CCH = 200
NCH = EPW // CCH
WCH = 104
NWB = N // WCH
WTAIL = N - NWB * WCH

E2 = E // 2
EPW2 = E2 // NW
NCH2 = EPW2 // CCH
SCCH = 200
NSCH = EPW // SCCH

_f32 = jnp.float32


@functools.lru_cache(maxsize=1)
def _sc_mesh():
    return plsc.VectorSubcoreMesh(
        core_axis_name="c", subcore_axis_name="s",
        num_cores=NC, num_subcores=NS)


def _ln(v, g, b):
    mu = jnp.mean(v, axis=1, keepdims=True)
    var = jnp.mean((v - mu) ** 2, axis=1, keepdims=True)
    return (v - mu) * jax.lax.rsqrt(var + 1e-5) * g + b


def _dot(a, b):
    return jnp.dot(a, b, preferred_element_type=_f32)


def _node_pre(h_ref, x_ref, qwT_ref, qb_ref, w1hT_ref, tabA_ref, tabB_ref):
    h = h_ref[...]
    hk = _dot(h, w1hT_ref[...])
    q = _dot(h, qwT_ref[...]) + qb_ref[...]
    xpad = jnp.concatenate(
        [x_ref[...], jnp.zeros((h.shape[0], TAB - HIDDEN - 3), _f32)], axis=1)
    tabA_ref[...] = jnp.concatenate([hk, xpad], axis=1)
    tabB_ref[...] = jnp.concatenate([q, xpad], axis=1)


def _run_node_pre(h, x, qwT, qb, w1hT):
    grid = (N // BN,)
    full = lambda shp: pl.BlockSpec(shp, lambda i: (0, 0))
    return pl.pallas_call(
        _node_pre,
        grid=grid,
        in_specs=[
            pl.BlockSpec((BN, HIDDEN), lambda i: (i, 0)),
            pl.BlockSpec((BN, 3), lambda i: (i, 0)),
            full((HIDDEN, HIDDEN)),
            full((1, HIDDEN)),
            full((HIDDEN, HIDDEN)),
        ],
        out_specs=[
            pl.BlockSpec((BN, TAB), lambda i: (i, 0)),
            pl.BlockSpec((BN, TAB), lambda i: (i, 0)),
        ],
        out_shape=[
            jax.ShapeDtypeStruct((N, TAB), _f32),
            jax.ShapeDtypeStruct((N, TAB), _f32),
        ],
    )(h, x, qwT, qb, w1hT)


def _edge_mlp(gA_ref, gB_ref, ea_ref, w1eT_ref, kvb1_ref, lng_ref, lnb_ref,
              w2T_ref, kvb2_ref, abw1T_ref, abb1_ref, ablng_ref, ablnb_ref,
              abw2T_ref, abb2_ref, combo_ref, v_ref):
    a = gA_ref[...]
    b = gB_ref[...]
    xs = a[:, HIDDEN:HIDDEN + 3]
    xd = b[:, HIDDEN:HIDDEN + 3]
    rel = xd - xs
    d2 = jnp.sum(rel * rel, axis=1, keepdims=True)
    d = jnp.sqrt(d2 + 1e-8)
    delta = CUTOFF / (NUM_RBF - 1)
    off = lax.broadcasted_iota(jnp.int32, (1, NUM_RBF), 1).astype(_f32) * delta
    df = jnp.exp((-0.5 / (delta * delta)) * (d - off) ** 2)
    ci = lax.broadcasted_iota(jnp.int32, (NUM_RBF, OUTER), 1)
    ri = lax.broadcasted_iota(jnp.int32, (NUM_RBF, OUTER), 0)
    Rm = (ci // EDGE_FEAT_DIM == ri).astype(_f32)
    cj = lax.broadcasted_iota(jnp.int32, (EDGE_FEAT_DIM, OUTER), 1)
    rj = lax.broadcasted_iota(jnp.int32, (EDGE_FEAT_DIM, OUTER), 0)
    Sm = (cj % EDGE_FEAT_DIM == rj).astype(_f32)
    ef = _dot(df, Rm) * _dot(ea_ref[...], Sm)
    t1 = a[:, :HIDDEN] + _dot(ef, w1eT_ref[...]) + kvb1_ref[...]
    t1 = jnp.maximum(_ln(t1, lng_ref[...], lnb_ref[...]), 0.0)
    kv = _dot(t1, w2T_ref[...]) + kvb2_ref[...]
    k = kv[:, :HIDDEN]
    v = kv[:, HIDDEN:]
    q = b[:, :HIDDEN]
    di = lax.broadcasted_iota(jnp.int32, (HIDDEN, N_HEADS), 0)
    hi = lax.broadcasted_iota(jnp.int32, (HIDDEN, N_HEADS), 1)
    Hm = (di // HEAD_DIM == hi).astype(_f32)
    logits = _dot(q * k, Hm) * SCALE
    tb = _dot(ef, abw1T_ref[...]) + abb1_ref[...]
    tb = jnp.maximum(_ln(tb, ablng_ref[...], ablnb_ref[...]), 0.0)
    ab = _dot(tb, abw2T_ref[...]) + abb2_ref[...]
    n = rel.shape[0]
    combo_ref[...] = jnp.concatenate(
        [jnp.exp(logits + ab), rel / (d + 1.0),
         jnp.zeros((n, HIDDEN - N_HEADS - 3), _f32)], axis=1)
    v_ref[...] = v


def _run_edge_mlp(gA, gB, ea, w1eT, kvb1, lng, lnb, w2T, kvb2,
                  abw1T, abb1, ablng, ablnb, abw2T, abb2):
    grid = (E2 // BE,)
    full = lambda shp: pl.BlockSpec(shp, lambda i: (0, 0))
    return pl.pallas_call(
        _edge_mlp,
        grid=grid,
        in_specs=[
            pl.BlockSpec((BE, TAB), lambda i: (i, 0)),
            pl.BlockSpec((BE, TAB), lambda i: (i, 0)),
            pl.BlockSpec((BE, EDGE_FEAT_DIM), lambda i: (i, 0)),
            full((OUTER, HIDDEN)),
            full((1, HIDDEN)),
            full((1, HIDDEN)),
            full((1, HIDDEN)),
            full((HIDDEN, 2 * HIDDEN)),
            full((1, 2 * HIDDEN)),
            full((OUTER, HIDDEN)),
            full((1, HIDDEN)),
            full((1, HIDDEN)),
            full((1, HIDDEN)),
            full((HIDDEN, N_HEADS)),
            full((1, N_HEADS)),
        ],
        out_specs=[
            pl.BlockSpec((BE, HIDDEN), lambda i: (i, 0)),
            pl.BlockSpec((BE, HIDDEN), lambda i: (i, 0)),
        ],
        out_shape=[
            jax.ShapeDtypeStruct((E2, HIDDEN), _f32),
            jax.ShapeDtypeStruct((E2, HIDDEN), _f32),
        ],
    )(gA, gB, ea, w1eT, kvb1, lng, lnb, w2T, kvb2,
      abw1T, abb1, ablng, ablnb, abw2T, abb2)


def _sum2(pa_ref, pb_ref, out_ref):
    out_ref[...] = pa_ref[0] + pb_ref[0]


def _run_sum2(parts):
    grid = (N // BN,)
    return pl.pallas_call(
        _sum2,
        grid=grid,
        in_specs=[
            pl.BlockSpec((1, BN, HIDDEN), lambda i: (0, i, 0)),
            pl.BlockSpec((1, BN, HIDDEN), lambda i: (1, i, 0)),
        ],
        out_specs=pl.BlockSpec((BN, HIDDEN), lambda i: (i, 0)),
        out_shape=jax.ShapeDtypeStruct((N, HIDDEN), _f32),
    )(parts, parts)


def _edge2(combo_ref, sg_ref, v_ref, xw1T_ref, xb1_ref, xw2_ref,
           msg_ref, vecp_ref):
    combo = combo_ref[...]
    w = combo[:, :N_HEADS] / (sg_ref[...][:, :N_HEADS] + 1e-16)
    relod = combo[:, N_HEADS:N_HEADS + 3]
    hi = lax.broadcasted_iota(jnp.int32, (N_HEADS, HIDDEN), 0)
    di = lax.broadcasted_iota(jnp.int32, (N_HEADS, HIDDEN), 1)
    Bm = (di // HEAD_DIM == hi).astype(_f32)
    msg = _dot(w, Bm) * v_ref[...]
    m1 = _dot(msg, xw1T_ref[...]) + xb1_ref[...]
    m1 = m1 * jax.nn.sigmoid(m1)
    cc = jnp.sum(m1 * xw2_ref[...], axis=1, keepdims=True)
    coef = jnp.tanh(cc)
    vec = relod * coef
    msg_ref[...] = msg
    vecp_ref[...] = jnp.concatenate(
        [vec, jnp.zeros((vec.shape[0], HIDDEN - 3), _f32)], axis=1)


def _run_edge2(combo, sg, v, xw1T, xb1, xw2):
    grid = (E2 // BE,)
    full = lambda shp: pl.BlockSpec(shp, lambda i: (0, 0))
    return pl.pallas_call(
        _edge2,
        grid=grid,
        in_specs=[
            pl.BlockSpec((BE, HIDDEN), lambda i: (i, 0)),
            pl.BlockSpec((BE, HIDDEN), lambda i: (i, 0)),
            pl.BlockSpec((BE, HIDDEN), lambda i: (i, 0)),
            full((HIDDEN, HIDDEN)),
            full((1, HIDDEN)),
            full((1, HIDDEN)),
        ],
        out_specs=[
            pl.BlockSpec((BE, HIDDEN), lambda i: (i, 0)),
            pl.BlockSpec((BE, HIDDEN), lambda i: (i, 0)),
        ],
        out_shape=[
            jax.ShapeDtypeStruct((E2, HIDDEN), _f32),
            jax.ShapeDtypeStruct((E2, HIDDEN), _f32),
        ],
    )(combo, sg, v, xw1T, xb1, xw2)


def _node_post(pa_ref, va_ref, h_ref, x_ref, mask_ref,
               outwT_ref, outb_ref, nmw1T_ref, nmb1_ref, nmlng_ref,
               nmlnb_ref, nmw2T_ref, nmb2_ref, hnew_ref, xnew_ref):
    agg = pa_ref[0]
    dxv = va_ref[0][:, :3]
    h = h_ref[...]
    aggo = _dot(agg, outwT_ref[...]) + outb_ref[...]
    tn = jnp.concatenate([aggo, h], axis=1)
    tn = _dot(tn, nmw1T_ref[...]) + nmb1_ref[...]
    tn = jnp.maximum(_ln(tn, nmlng_ref[...], nmlnb_ref[...]), 0.0)
    tn = _dot(tn, nmw2T_ref[...]) + nmb2_ref[...]
    hnew_ref[...] = h + tn
    xnew_ref[...] = x_ref[...] + dxv * mask_ref[...]


def _run_node_post(parts, h, x, maskf, outwT, outb, nmw1T, nmb1,
                   nmlng, nmlnb, nmw2T, nmb2):
    grid = (N // BN,)
    full = lambda shp: pl.BlockSpec(shp, lambda i: (0, 0))
    return pl.pallas_call(
        _node_post,
        grid=grid,
        in_specs=[
            pl.BlockSpec((1, BN, HIDDEN), lambda i: (0, i, 0)),
            pl.BlockSpec((1, BN, HIDDEN), lambda i: (1, i, 0)),
            pl.BlockSpec((BN, HIDDEN), lambda i: (i, 0)),
            pl.BlockSpec((BN, 3), lambda i: (i, 0)),
            pl.BlockSpec((BN, 1), lambda i: (i, 0)),
            full((HIDDEN, HIDDEN)),
            full((1, HIDDEN)),
            full((2 * HIDDEN, HIDDEN)),
            full((1, HIDDEN)),
            full((1, HIDDEN)),
            full((1, HIDDEN)),
            full((HIDDEN, HIDDEN)),
            full((1, HIDDEN)),
        ],
        out_specs=[
            pl.BlockSpec((BN, HIDDEN), lambda i: (i, 0)),
            pl.BlockSpec((BN, 3), lambda i: (i, 0)),
        ],
        out_shape=[
            jax.ShapeDtypeStruct((N, HIDDEN), _f32),
            jax.ShapeDtypeStruct((N, 3), _f32),
        ],
    )(parts, parts, h, x, maskf, outwT, outb, nmw1T, nmb1,
      nmlng, nmlnb, nmw2T, nmb2)


def _sc_gather2_body(tabA, tabB, src, dst, gA, gB,
                     idxa, idxb, bufa, bufb, sema, semb):
    wid = lax.axis_index("s") * NC + lax.axis_index("c")
    base0 = wid * EPW2

    def chunk(c, carry):
        base = base0 + c * CCH
        pltpu.sync_copy(src.at[pl.ds(base, CCH)], idxa)
        cpa = pltpu.async_copy(tabA.at[idxa], bufa, sema)
        pltpu.sync_copy(dst.at[pl.ds(base, CCH)], idxb)
        cpb = pltpu.async_copy(tabB.at[idxb], bufb, semb)
        cpa.wait()
        pltpu.sync_copy(bufa, gA.at[pl.ds(base, CCH), :])
        cpb.wait()
        pltpu.sync_copy(bufb, gB.at[pl.ds(base, CCH), :])
        return carry

    lax.fori_loop(0, NCH2, chunk, 0)


def _run_sc_gather2(tabA, tabB, src, dst):
    return pl.kernel(
        _sc_gather2_body,
        out_type=[
            jax.ShapeDtypeStruct((E2, TAB), _f32),
            jax.ShapeDtypeStruct((E2, TAB), _f32),
        ],
        mesh=_sc_mesh(),
        scratch_types=[
            pltpu.VMEM((CCH,), jnp.int32),
            pltpu.VMEM((CCH,), jnp.int32),
            pltpu.VMEM((CCH, TAB), _f32),
            pltpu.VMEM((CCH, TAB), _f32),
            pltpu.SemaphoreType.DMA,
            pltpu.SemaphoreType.DMA,
        ],
    )(tabA, tabB, src, dst)


def _acc_zero(zrows, acc_ref, sid):
    for w in range(NWB // NS):
        r = (sid * (NWB // NS) + w) * WCH
        pltpu.sync_copy(zrows.at[pl.ds(r, WCH), :],
                        acc_ref.at[pl.ds(r, WCH), :])

    @pl.when(sid == 0)
    def _():
        pltpu.sync_copy(zrows.at[pl.ds(NWB * WCH, WTAIL), :],
                        acc_ref.at[pl.ds(NWB * WCH, WTAIL), :])


def _acc_writeback(acc_ref, out, cid, sid, wbuf, tbuf):
    for w in range(NWB // NS):
        r = (sid * (NWB // NS) + w) * WCH
        pltpu.sync_copy(acc_ref.at[pl.ds(r, WCH), :], wbuf)
        pltpu.sync_copy(wbuf, out.at[cid, pl.ds(r, WCH), :])

    @pl.when(sid == 0)
    def _():
        pltpu.sync_copy(acc_ref.at[pl.ds(NWB * WCH, WTAIL), :], tbuf)
        pltpu.sync_copy(tbuf, out.at[cid, pl.ds(NWB * WCH, WTAIL), :])


def _scatter_loop(vals, dst, acc_ref, idxv, bufv, base0, nchunks):
    def chunk(c, carry):
        base = base0 + c * SCCH
        pltpu.sync_copy(dst.at[pl.ds(base, SCCH)], idxv)
        pltpu.sync_copy(vals.at[pl.ds(base, SCCH), :], bufv)
        pltpu.sync_copy(bufv, acc_ref.at[idxv], add=True)
        return carry

    lax.fori_loop(0, nchunks, chunk, 0)


def _sc_scatter_part_body(vals_a, vals_b, dst_a, dst_b, zrows, out, idxv,
                          bufv, wbuf, tbuf, acc_ref):
    cid = lax.axis_index("c")
    sid = lax.axis_index("s")
    _acc_zero(zrows, acc_ref, sid)
    plsc.subcore_barrier()
    base0 = sid * EPW

    @pl.when(cid == 0)
    def _():
        _scatter_loop(vals_a, dst_a, acc_ref, idxv, bufv, base0, NSCH)

    @pl.when(cid == 1)
    def _():
        _scatter_loop(vals_b, dst_b, acc_ref, idxv, bufv, base0, NSCH)

    plsc.subcore_barrier()
    _acc_writeback(acc_ref, out, cid, sid, wbuf, tbuf)


def _run_sc_scatter_part(vals_a, vals_b, dst_a, dst_b, zrows):
    return pl.kernel(
        _sc_scatter_part_body,
        out_type=jax.ShapeDtypeStruct((NC, N, HIDDEN), _f32),
        mesh=_sc_mesh(),
        scratch_types=[
            pltpu.VMEM((SCCH,), jnp.int32),
            pltpu.VMEM((SCCH, HIDDEN), _f32),
            pltpu.VMEM((WCH, HIDDEN), _f32),
            pltpu.VMEM((WTAIL, HIDDEN), _f32),
            pltpu.VMEM_SHARED((N, HIDDEN), _f32),
        ],
    )(vals_a, vals_b, dst_a, dst_b, zrows)


def _sc_scatter_dual_body(ma, mb, va, vb, dst_a, dst_b, zrows, out, idxv,
                          bufv, wbuf, tbuf, acc_ref):
    cid = lax.axis_index("c")
    sid = lax.axis_index("s")
    _acc_zero(zrows, acc_ref, sid)
    plsc.subcore_barrier()
    base0 = sid * EPW

    @pl.when(cid == 0)
    def _():
        _scatter_loop(ma, dst_a, acc_ref, idxv, bufv, base0, NSCH)
        _scatter_loop(mb, dst_b, acc_ref, idxv, bufv, base0, NSCH)

    @pl.when(cid == 1)
    def _():
        _scatter_loop(va, dst_a, acc_ref, idxv, bufv, base0, NSCH)
        _scatter_loop(vb, dst_b, acc_ref, idxv, bufv, base0, NSCH)

    plsc.subcore_barrier()
    _acc_writeback(acc_ref, out, cid, sid, wbuf, tbuf)


def _run_sc_scatter_dual(ma, mb, va, vb, dst_a, dst_b, zrows):
    return pl.kernel(
        _sc_scatter_dual_body,
        out_type=jax.ShapeDtypeStruct((NC, N, HIDDEN), _f32),
        mesh=_sc_mesh(),
        scratch_types=[
            pltpu.VMEM((SCCH,), jnp.int32),
            pltpu.VMEM((SCCH, HIDDEN), _f32),
            pltpu.VMEM((WCH, HIDDEN), _f32),
            pltpu.VMEM((WTAIL, HIDDEN), _f32),
            pltpu.VMEM_SHARED((N, HIDDEN), _f32),
        ],
    )(ma, mb, va, vb, dst_a, dst_b, zrows)


def _sc_gather1_body(tab, dst, out, idxa, idxb, bufa, bufb, sema, semb):
    wid = lax.axis_index("s") * NC + lax.axis_index("c")
    base0 = wid * EPW2

    def pair(c, carry):
        base = base0 + c * 2 * CCH
        pltpu.sync_copy(dst.at[pl.ds(base, CCH)], idxa)
        cpa = pltpu.async_copy(tab.at[idxa], bufa, sema)
        pltpu.sync_copy(dst.at[pl.ds(base + CCH, CCH)], idxb)
        cpb = pltpu.async_copy(tab.at[idxb], bufb, semb)
        cpa.wait()
        pltpu.sync_copy(bufa, out.at[pl.ds(base, CCH), :])
        cpb.wait()
        pltpu.sync_copy(bufb, out.at[pl.ds(base + CCH, CCH), :])
        return carry

    lax.fori_loop(0, NCH2 // 2, pair, 0)
    base = base0 + (NCH2 - 1) * CCH
    pltpu.sync_copy(dst.at[pl.ds(base, CCH)], idxa)
    pltpu.async_copy(tab.at[idxa], bufa, sema).wait()
    pltpu.sync_copy(bufa, out.at[pl.ds(base, CCH), :])


def _run_sc_gather1(tab, dst):
    return pl.kernel(
        _sc_gather1_body,
        out_type=jax.ShapeDtypeStruct((E2, HIDDEN), _f32),
        mesh=_sc_mesh(),
        scratch_types=[
            pltpu.VMEM((CCH,), jnp.int32),
            pltpu.VMEM((CCH,), jnp.int32),
            pltpu.VMEM((CCH, HIDDEN), _f32),
            pltpu.VMEM((CCH, HIDDEN), _f32),
            pltpu.SemaphoreType.DMA,
            pltpu.SemaphoreType.DMA,
        ],
    )(tab, dst)


def kernel(h, x, edge_index, mask_ligand, edge_attr, params):
    p = params
    row = lambda v: v.reshape(1, -1)
    qwT = p["q_w"].T
    w1hT = p["kv_w1"][:, :HIDDEN].T
    w1eT = p["kv_w1"][:, HIDDEN:].T
    w2T = p["kv_w2"].T
    abw1T = p["ab_w1"].T
    abw2T = p["ab_w2"].T
    outwT = p["out_w"].T
    nmw1T = p["nm_w1"].T
    nmw2T = p["nm_w2"].T
    xw1T = p["x_w1"].T
    maskf = mask_ligand.astype(_f32).reshape(N, 1)
    src = edge_index[0]
    dst = edge_index[1]

    tabA, tabB = _run_node_pre(h, x, qwT, row(p["q_b"]), w1hT)

    zrows = jnp.zeros((N, HIDDEN), _f32)
    src_a, src_b = src[:E2], src[E2:]
    dst_a, dst_b = dst[:E2], dst[E2:]
    ea_a, ea_b = edge_attr[:E2], edge_attr[E2:]

    gA_a, gB_a = _run_sc_gather2(tabA, tabB, src_a, dst_a)
    gA_b, gB_b = _run_sc_gather2(tabA, tabB, src_b, dst_b)

    def mlp(gA, gB, ea):
        return _run_edge_mlp(
            gA, gB, ea, w1eT, row(p["kv_b1"]), row(p["kv_ln_g"]),
            row(p["kv_ln_b"]), w2T, row(p["kv_b2"]), abw1T, row(p["ab_b1"]),
            row(p["ab_ln_g"]), row(p["ab_ln_b"]), abw2T, row(p["ab_b2"]))

    combo_a, v_a = mlp(gA_a, gB_a, ea_a)
    combo_b, v_b = mlp(gA_b, gB_b, ea_b)

    s_parts = _run_sc_scatter_part(combo_a, combo_b, dst_a, dst_b, zrows)
    s = _run_sum2(s_parts)
    sg_a = _run_sc_gather1(s, dst_a)
    sg_b = _run_sc_gather1(s, dst_b)

    xw2r = p["x_w2"].reshape(1, HIDDEN)
    msg_a, vecp_a = _run_edge2(combo_a, sg_a, v_a, xw1T, row(p["x_b1"]), xw2r)
    msg_b, vecp_b = _run_edge2(combo_b, sg_b, v_b, xw1T, row(p["x_b1"]), xw2r)

    parts = _run_sc_scatter_dual(msg_a, msg_b, vecp_a, vecp_b,
                                 dst_a, dst_b, zrows)

    h_new, x_new = _run_node_post(
        parts, h, x, maskf, outwT, row(p["out_b"]), nmw1T, row(p["nm_b1"]),
        row(p["nm_ln_g"]), row(p["nm_ln_b"]), nmw2T, row(p["nm_b2"]))
    return (h_new, x_new)


# speedup vs baseline: 4.4176x; 1.0776x over previous
import functools
import math

import jax
import jax.numpy as jnp
from jax import lax
from jax.experimental import pallas as pl
from jax.experimental.pallas import tpu as pltpu
from jax.experimental.pallas import tpu_sc as plsc

N = 10000
E = 320000
HIDDEN = 128
EDGE_FEAT_DIM = 4
NUM_RBF = 16
N_HEADS = 16
HEAD_DIM = HIDDEN // N_HEADS
CUTOFF = 10.0
SCALE = 1.0 / math.sqrt(HEAD_DIM)
OUTER = NUM_RBF * EDGE_FEAT_DIM

TAB = 256
BN = 1000
BE = 2000

NC = 2
NS = 16
NW = NC * NS
EPW = E // NW
CCH = 200
NCH = EPW // CCH
WCH = 104
NWB = N // WCH
WTAIL = N - NWB * WCH

E2 = E // 2
EPW2 = E2 // NW
NCH2 = EPW2 // CCH
SCCH = 200
NSCH = EPW // SCCH

_f32 = jnp.float32


@functools.lru_cache(maxsize=1)
def _sc_mesh():
    return plsc.VectorSubcoreMesh(
        core_axis_name="c", subcore_axis_name="s",
        num_cores=NC, num_subcores=NS)


def _ln(v, g, b):
    mu = jnp.mean(v, axis=1, keepdims=True)
    var = jnp.mean((v - mu) ** 2, axis=1, keepdims=True)
    return (v - mu) * jax.lax.rsqrt(var + 1e-5) * g + b


def _dot(a, b):
    return jnp.dot(a, b, preferred_element_type=_f32)


def _node_pre(h_ref, x_ref, qwT_ref, qb_ref, w1hT_ref, tabA_ref, tabB_ref):
    h = h_ref[...]
    hk = _dot(h, w1hT_ref[...])
    q = _dot(h, qwT_ref[...]) + qb_ref[...]
    xpad = jnp.concatenate(
        [x_ref[...], jnp.zeros((h.shape[0], TAB - HIDDEN - 3), _f32)], axis=1)
    tabA_ref[...] = jnp.concatenate([hk, xpad], axis=1)
    tabB_ref[...] = jnp.concatenate([q, xpad], axis=1)


def _run_node_pre(h, x, qwT, qb, w1hT):
    grid = (N // BN,)
    full = lambda shp: pl.BlockSpec(shp, lambda i: (0, 0))
    return pl.pallas_call(
        _node_pre,
        grid=grid,
        in_specs=[
            pl.BlockSpec((BN, HIDDEN), lambda i: (i, 0)),
            pl.BlockSpec((BN, 3), lambda i: (i, 0)),
            full((HIDDEN, HIDDEN)),
            full((1, HIDDEN)),
            full((HIDDEN, HIDDEN)),
        ],
        out_specs=[
            pl.BlockSpec((BN, TAB), lambda i: (i, 0)),
            pl.BlockSpec((BN, TAB), lambda i: (i, 0)),
        ],
        out_shape=[
            jax.ShapeDtypeStruct((N, TAB), _f32),
            jax.ShapeDtypeStruct((N, TAB), _f32),
        ],
    )(h, x, qwT, qb, w1hT)


def _edge_mlp(gA_ref, gB_ref, ea_ref, w1eT_ref, kvb1_ref, lng_ref, lnb_ref,
              w2T_ref, kvb2_ref, abw1T_ref, abb1_ref, ablng_ref, ablnb_ref,
              abw2T_ref, abb2_ref, combo_ref, v_ref):
    a = gA_ref[...]
    b = gB_ref[...]
    xs = a[:, HIDDEN:HIDDEN + 3]
    xd = b[:, HIDDEN:HIDDEN + 3]
    rel = xd - xs
    d2 = jnp.sum(rel * rel, axis=1, keepdims=True)
    d = jnp.sqrt(d2 + 1e-8)
    delta = CUTOFF / (NUM_RBF - 1)
    off = lax.broadcasted_iota(jnp.int32, (1, NUM_RBF), 1).astype(_f32) * delta
    df = jnp.exp((-0.5 / (delta * delta)) * (d - off) ** 2)
    ci = lax.broadcasted_iota(jnp.int32, (NUM_RBF, OUTER), 1)
    ri = lax.broadcasted_iota(jnp.int32, (NUM_RBF, OUTER), 0)
    Rm = (ci // EDGE_FEAT_DIM == ri).astype(_f32)
    cj = lax.broadcasted_iota(jnp.int32, (EDGE_FEAT_DIM, OUTER), 1)
    rj = lax.broadcasted_iota(jnp.int32, (EDGE_FEAT_DIM, OUTER), 0)
    Sm = (cj % EDGE_FEAT_DIM == rj).astype(_f32)
    ef = _dot(df, Rm) * _dot(ea_ref[...], Sm)
    t1 = a[:, :HIDDEN] + _dot(ef, w1eT_ref[...]) + kvb1_ref[...]
    t1 = jnp.maximum(_ln(t1, lng_ref[...], lnb_ref[...]), 0.0)
    kv = _dot(t1, w2T_ref[...]) + kvb2_ref[...]
    k = kv[:, :HIDDEN]
    v = kv[:, HIDDEN:]
    q = b[:, :HIDDEN]
    di = lax.broadcasted_iota(jnp.int32, (HIDDEN, N_HEADS), 0)
    hi = lax.broadcasted_iota(jnp.int32, (HIDDEN, N_HEADS), 1)
    Hm = (di // HEAD_DIM == hi).astype(_f32)
    logits = _dot(q * k, Hm) * SCALE
    tb = _dot(ef, abw1T_ref[...]) + abb1_ref[...]
    tb = jnp.maximum(_ln(tb, ablng_ref[...], ablnb_ref[...]), 0.0)
    ab = _dot(tb, abw2T_ref[...]) + abb2_ref[...]
    n = rel.shape[0]
    combo_ref[...] = jnp.concatenate(
        [jnp.exp(logits + ab), rel / (d + 1.0),
         jnp.zeros((n, HIDDEN - N_HEADS - 3), _f32)], axis=1)
    v_ref[...] = v


def _run_edge_mlp(gA, gB, ea, w1eT, kvb1, lng, lnb, w2T, kvb2,
                  abw1T, abb1, ablng, ablnb, abw2T, abb2):
    grid = (E2 // BE,)
    full = lambda shp: pl.BlockSpec(shp, lambda i: (0, 0))
    return pl.pallas_call(
        _edge_mlp,
        grid=grid,
        in_specs=[
            pl.BlockSpec((BE, TAB), lambda i: (i, 0)),
            pl.BlockSpec((BE, TAB), lambda i: (i, 0)),
            pl.BlockSpec((BE, EDGE_FEAT_DIM), lambda i: (i, 0)),
            full((OUTER, HIDDEN)),
            full((1, HIDDEN)),
            full((1, HIDDEN)),
            full((1, HIDDEN)),
            full((HIDDEN, 2 * HIDDEN)),
            full((1, 2 * HIDDEN)),
            full((OUTER, HIDDEN)),
            full((1, HIDDEN)),
            full((1, HIDDEN)),
            full((1, HIDDEN)),
            full((HIDDEN, N_HEADS)),
            full((1, N_HEADS)),
        ],
        out_specs=[
            pl.BlockSpec((BE, HIDDEN), lambda i: (i, 0)),
            pl.BlockSpec((BE, HIDDEN), lambda i: (i, 0)),
        ],
        out_shape=[
            jax.ShapeDtypeStruct((E2, HIDDEN), _f32),
            jax.ShapeDtypeStruct((E2, HIDDEN), _f32),
        ],
    )(gA, gB, ea, w1eT, kvb1, lng, lnb, w2T, kvb2,
      abw1T, abb1, ablng, ablnb, abw2T, abb2)


def _sum2(pa_ref, pb_ref, out_ref):
    out_ref[...] = pa_ref[0] + pb_ref[0]


def _run_sum2(parts):
    grid = (N // BN,)
    return pl.pallas_call(
        _sum2,
        grid=grid,
        in_specs=[
            pl.BlockSpec((1, BN, HIDDEN), lambda i: (0, i, 0)),
            pl.BlockSpec((1, BN, HIDDEN), lambda i: (1, i, 0)),
        ],
        out_specs=pl.BlockSpec((BN, HIDDEN), lambda i: (i, 0)),
        out_shape=jax.ShapeDtypeStruct((N, HIDDEN), _f32),
    )(parts, parts)


def _edge2(combo_ref, sg_ref, v_ref, xw1T_ref, xb1_ref, xw2_ref,
           msg_ref, vecp_ref):
    combo = combo_ref[...]
    w = combo[:, :N_HEADS] / (sg_ref[...][:, :N_HEADS] + 1e-16)
    relod = combo[:, N_HEADS:N_HEADS + 3]
    hi = lax.broadcasted_iota(jnp.int32, (N_HEADS, HIDDEN), 0)
    di = lax.broadcasted_iota(jnp.int32, (N_HEADS, HIDDEN), 1)
    Bm = (di // HEAD_DIM == hi).astype(_f32)
    msg = _dot(w, Bm) * v_ref[...]
    m1 = _dot(msg, xw1T_ref[...]) + xb1_ref[...]
    m1 = m1 * jax.nn.sigmoid(m1)
    cc = jnp.sum(m1 * xw2_ref[...], axis=1, keepdims=True)
    coef = jnp.tanh(cc)
    vec = relod * coef
    msg_ref[...] = msg
    vecp_ref[...] = jnp.concatenate(
        [vec, jnp.zeros((vec.shape[0], HIDDEN - 3), _f32)], axis=1)


def _run_edge2(combo, sg, v, xw1T, xb1, xw2):
    grid = (E2 // BE,)
    full = lambda shp: pl.BlockSpec(shp, lambda i: (0, 0))
    return pl.pallas_call(
        _edge2,
        grid=grid,
        in_specs=[
            pl.BlockSpec((BE, HIDDEN), lambda i: (i, 0)),
            pl.BlockSpec((BE, HIDDEN), lambda i: (i, 0)),
            pl.BlockSpec((BE, HIDDEN), lambda i: (i, 0)),
            full((HIDDEN, HIDDEN)),
            full((1, HIDDEN)),
            full((1, HIDDEN)),
        ],
        out_specs=[
            pl.BlockSpec((BE, HIDDEN), lambda i: (i, 0)),
            pl.BlockSpec((BE, HIDDEN), lambda i: (i, 0)),
        ],
        out_shape=[
            jax.ShapeDtypeStruct((E2, HIDDEN), _f32),
            jax.ShapeDtypeStruct((E2, HIDDEN), _f32),
        ],
    )(combo, sg, v, xw1T, xb1, xw2)


def _node_post(pa_ref, pb_ref, va_ref, vb_ref, h_ref, x_ref, mask_ref,
               outwT_ref, outb_ref, nmw1T_ref, nmb1_ref, nmlng_ref,
               nmlnb_ref, nmw2T_ref, nmb2_ref, hnew_ref, xnew_ref):
    agg = pa_ref[0] + pb_ref[0]
    dxv = (va_ref[0] + vb_ref[0])[:, :3]
    h = h_ref[...]
    aggo = _dot(agg, outwT_ref[...]) + outb_ref[...]
    tn = jnp.concatenate([aggo, h], axis=1)
    tn = _dot(tn, nmw1T_ref[...]) + nmb1_ref[...]
    tn = jnp.maximum(_ln(tn, nmlng_ref[...], nmlnb_ref[...]), 0.0)
    tn = _dot(tn, nmw2T_ref[...]) + nmb2_ref[...]
    hnew_ref[...] = h + tn
    xnew_ref[...] = x_ref[...] + dxv * mask_ref[...]


def _run_node_post(parts_a, parts_b, h, x, maskf, outwT, outb, nmw1T, nmb1,
                   nmlng, nmlnb, nmw2T, nmb2):
    grid = (N // BN,)
    full = lambda shp: pl.BlockSpec(shp, lambda i: (0, 0))
    return pl.pallas_call(
        _node_post,
        grid=grid,
        in_specs=[
            pl.BlockSpec((1, BN, HIDDEN), lambda i: (0, i, 0)),
            pl.BlockSpec((1, BN, HIDDEN), lambda i: (0, i, 0)),
            pl.BlockSpec((1, BN, HIDDEN), lambda i: (1, i, 0)),
            pl.BlockSpec((1, BN, HIDDEN), lambda i: (1, i, 0)),
            pl.BlockSpec((BN, HIDDEN), lambda i: (i, 0)),
            pl.BlockSpec((BN, 3), lambda i: (i, 0)),
            pl.BlockSpec((BN, 1), lambda i: (i, 0)),
            full((HIDDEN, HIDDEN)),
            full((1, HIDDEN)),
            full((2 * HIDDEN, HIDDEN)),
            full((1, HIDDEN)),
            full((1, HIDDEN)),
            full((1, HIDDEN)),
            full((HIDDEN, HIDDEN)),
            full((1, HIDDEN)),
        ],
        out_specs=[
            pl.BlockSpec((BN, HIDDEN), lambda i: (i, 0)),
            pl.BlockSpec((BN, 3), lambda i: (i, 0)),
        ],
        out_shape=[
            jax.ShapeDtypeStruct((N, HIDDEN), _f32),
            jax.ShapeDtypeStruct((N, 3), _f32),
        ],
    )(parts_a, parts_b, parts_a, parts_b, h, x, maskf, outwT, outb,
      nmw1T, nmb1, nmlng, nmlnb, nmw2T, nmb2)


def _sc_gather2_body(tabA, tabB, src, dst, gA, gB,
                     idxa, idxb, bufa, bufb, sema, semb):
    wid = lax.axis_index("s") * NC + lax.axis_index("c")
    base0 = wid * EPW2

    def chunk(c, carry):
        base = base0 + c * CCH
        pltpu.sync_copy(src.at[pl.ds(base, CCH)], idxa)
        cpa = pltpu.async_copy(tabA.at[idxa], bufa, sema)
        pltpu.sync_copy(dst.at[pl.ds(base, CCH)], idxb)
        cpb = pltpu.async_copy(tabB.at[idxb], bufb, semb)
        cpa.wait()
        pltpu.sync_copy(bufa, gA.at[pl.ds(base, CCH), :])
        cpb.wait()
        pltpu.sync_copy(bufb, gB.at[pl.ds(base, CCH), :])
        return carry

    lax.fori_loop(0, NCH2, chunk, 0)


def _run_sc_gather2(tabA, tabB, src, dst):
    return pl.kernel(
        _sc_gather2_body,
        out_type=[
            jax.ShapeDtypeStruct((E2, TAB), _f32),
            jax.ShapeDtypeStruct((E2, TAB), _f32),
        ],
        mesh=_sc_mesh(),
        scratch_types=[
            pltpu.VMEM((CCH,), jnp.int32),
            pltpu.VMEM((CCH,), jnp.int32),
            pltpu.VMEM((CCH, TAB), _f32),
            pltpu.VMEM((CCH, TAB), _f32),
            pltpu.SemaphoreType.DMA,
            pltpu.SemaphoreType.DMA,
        ],
    )(tabA, tabB, src, dst)


def _acc_zero(zrows, acc_ref, sid):
    for w in range(NWB // NS):
        r = (sid * (NWB // NS) + w) * WCH
        pltpu.sync_copy(zrows.at[pl.ds(r, WCH), :],
                        acc_ref.at[pl.ds(r, WCH), :])

    @pl.when(sid == 0)
    def _():
        pltpu.sync_copy(zrows.at[pl.ds(NWB * WCH, WTAIL), :],
                        acc_ref.at[pl.ds(NWB * WCH, WTAIL), :])


def _acc_writeback(acc_ref, out, cid, sid, wbuf, tbuf):
    for w in range(NWB // NS):
        r = (sid * (NWB // NS) + w) * WCH
        pltpu.sync_copy(acc_ref.at[pl.ds(r, WCH), :], wbuf)
        pltpu.sync_copy(wbuf, out.at[cid, pl.ds(r, WCH), :])

    @pl.when(sid == 0)
    def _():
        pltpu.sync_copy(acc_ref.at[pl.ds(NWB * WCH, WTAIL), :], tbuf)
        pltpu.sync_copy(tbuf, out.at[cid, pl.ds(NWB * WCH, WTAIL), :])


def _scatter_loop(vals, dst, acc_ref, idxv, bufv, base0, nchunks):
    def chunk(c, carry):
        base = base0 + c * SCCH
        pltpu.sync_copy(dst.at[pl.ds(base, SCCH)], idxv)
        pltpu.sync_copy(vals.at[pl.ds(base, SCCH), :], bufv)
        pltpu.sync_copy(bufv, acc_ref.at[idxv], add=True)
        return carry

    lax.fori_loop(0, nchunks, chunk, 0)


def _sc_scatter_part_body(vals_a, vals_b, dst_a, dst_b, zrows, out, idxv,
                          bufv, wbuf, tbuf, acc_ref):
    cid = lax.axis_index("c")
    sid = lax.axis_index("s")
    _acc_zero(zrows, acc_ref, sid)
    plsc.subcore_barrier()
    base0 = sid * EPW

    @pl.when(cid == 0)
    def _():
        _scatter_loop(vals_a, dst_a, acc_ref, idxv, bufv, base0, NSCH)

    @pl.when(cid == 1)
    def _():
        _scatter_loop(vals_b, dst_b, acc_ref, idxv, bufv, base0, NSCH)

    plsc.subcore_barrier()
    _acc_writeback(acc_ref, out, cid, sid, wbuf, tbuf)


def _run_sc_scatter_part(vals_a, vals_b, dst_a, dst_b, zrows):
    return pl.kernel(
        _sc_scatter_part_body,
        out_type=jax.ShapeDtypeStruct((NC, N, HIDDEN), _f32),
        mesh=_sc_mesh(),
        scratch_types=[
            pltpu.VMEM((SCCH,), jnp.int32),
            pltpu.VMEM((SCCH, HIDDEN), _f32),
            pltpu.VMEM((WCH, HIDDEN), _f32),
            pltpu.VMEM((WTAIL, HIDDEN), _f32),
            pltpu.VMEM_SHARED((N, HIDDEN), _f32),
        ],
    )(vals_a, vals_b, dst_a, dst_b, zrows)


def _sc_scatter_dual_body(ma, va, dst_h, zrows, out, idxv,
                          bufv, wbuf, tbuf, acc_ref):
    cid = lax.axis_index("c")
    sid = lax.axis_index("s")
    _acc_zero(zrows, acc_ref, sid)
    plsc.subcore_barrier()
    base0 = sid * EPW

    @pl.when(cid == 0)
    def _():
        _scatter_loop(ma, dst_h, acc_ref, idxv, bufv, base0, NSCH)

    @pl.when(cid == 1)
    def _():
        _scatter_loop(va, dst_h, acc_ref, idxv, bufv, base0, NSCH)

    plsc.subcore_barrier()
    _acc_writeback(acc_ref, out, cid, sid, wbuf, tbuf)


def _run_sc_scatter_dual(ma, va, dst_h, zrows):
    return pl.kernel(
        _sc_scatter_dual_body,
        out_type=jax.ShapeDtypeStruct((NC, N, HIDDEN), _f32),
        mesh=_sc_mesh(),
        scratch_types=[
            pltpu.VMEM((SCCH,), jnp.int32),
            pltpu.VMEM((SCCH, HIDDEN), _f32),
            pltpu.VMEM((WCH, HIDDEN), _f32),
            pltpu.VMEM((WTAIL, HIDDEN), _f32),
            pltpu.VMEM_SHARED((N, HIDDEN), _f32),
        ],
    )(ma, va, dst_h, zrows)


def _sc_gather1_body(tab, dst, out, idxa, idxb, bufa, bufb, sema, semb):
    wid = lax.axis_index("s") * NC + lax.axis_index("c")
    base0 = wid * EPW2

    def pair(c, carry):
        base = base0 + c * 2 * CCH
        pltpu.sync_copy(dst.at[pl.ds(base, CCH)], idxa)
        cpa = pltpu.async_copy(tab.at[idxa], bufa, sema)
        pltpu.sync_copy(dst.at[pl.ds(base + CCH, CCH)], idxb)
        cpb = pltpu.async_copy(tab.at[idxb], bufb, semb)
        cpa.wait()
        pltpu.sync_copy(bufa, out.at[pl.ds(base, CCH), :])
        cpb.wait()
        pltpu.sync_copy(bufb, out.at[pl.ds(base + CCH, CCH), :])
        return carry

    lax.fori_loop(0, NCH2 // 2, pair, 0)
    base = base0 + (NCH2 - 1) * CCH
    pltpu.sync_copy(dst.at[pl.ds(base, CCH)], idxa)
    pltpu.async_copy(tab.at[idxa], bufa, sema).wait()
    pltpu.sync_copy(bufa, out.at[pl.ds(base, CCH), :])


def _run_sc_gather1(tab, dst):
    return pl.kernel(
        _sc_gather1_body,
        out_type=jax.ShapeDtypeStruct((E2, HIDDEN), _f32),
        mesh=_sc_mesh(),
        scratch_types=[
            pltpu.VMEM((CCH,), jnp.int32),
            pltpu.VMEM((CCH,), jnp.int32),
            pltpu.VMEM((CCH, HIDDEN), _f32),
            pltpu.VMEM((CCH, HIDDEN), _f32),
            pltpu.SemaphoreType.DMA,
            pltpu.SemaphoreType.DMA,
        ],
    )(tab, dst)


def kernel(h, x, edge_index, mask_ligand, edge_attr, params):
    p = params
    row = lambda v: v.reshape(1, -1)
    qwT = p["q_w"].T
    w1hT = p["kv_w1"][:, :HIDDEN].T
    w1eT = p["kv_w1"][:, HIDDEN:].T
    w2T = p["kv_w2"].T
    abw1T = p["ab_w1"].T
    abw2T = p["ab_w2"].T
    outwT = p["out_w"].T
    nmw1T = p["nm_w1"].T
    nmw2T = p["nm_w2"].T
    xw1T = p["x_w1"].T
    maskf = mask_ligand.astype(_f32).reshape(N, 1)
    src = edge_index[0]
    dst = edge_index[1]

    tabA, tabB = _run_node_pre(h, x, qwT, row(p["q_b"]), w1hT)

    zrows = jnp.zeros((N, HIDDEN), _f32)
    src_a, src_b = src[:E2], src[E2:]
    dst_a, dst_b = dst[:E2], dst[E2:]
    ea_a, ea_b = edge_attr[:E2], edge_attr[E2:]

    gA_a, gB_a = _run_sc_gather2(tabA, tabB, src_a, dst_a)
    gA_b, gB_b = _run_sc_gather2(tabA, tabB, src_b, dst_b)

    def mlp(gA, gB, ea):
        return _run_edge_mlp(
            gA, gB, ea, w1eT, row(p["kv_b1"]), row(p["kv_ln_g"]),
            row(p["kv_ln_b"]), w2T, row(p["kv_b2"]), abw1T, row(p["ab_b1"]),
            row(p["ab_ln_g"]), row(p["ab_ln_b"]), abw2T, row(p["ab_b2"]))

    combo_a, v_a = mlp(gA_a, gB_a, ea_a)
    combo_b, v_b = mlp(gA_b, gB_b, ea_b)

    s_parts = _run_sc_scatter_part(combo_a, combo_b, dst_a, dst_b, zrows)
    s = _run_sum2(s_parts)
    sg_a = _run_sc_gather1(s, dst_a)
    sg_b = _run_sc_gather1(s, dst_b)

    xw2r = p["x_w2"].reshape(1, HIDDEN)
    msg_a, vecp_a = _run_edge2(combo_a, sg_a, v_a, xw1T, row(p["x_b1"]), xw2r)
    msg_b, vecp_b = _run_edge2(combo_b, sg_b, v_b, xw1T, row(p["x_b1"]), xw2r)

    parts_a = _run_sc_scatter_dual(msg_a, vecp_a, dst_a, zrows)
    parts_b = _run_sc_scatter_dual(msg_b, vecp_b, dst_b, zrows)

    h_new, x_new = _run_node_post(
        parts_a, parts_b, h, x, maskf, outwT, row(p["out_b"]), nmw1T,
        row(p["nm_b1"]), row(p["nm_ln_g"]), row(p["nm_ln_b"]), nmw2T,
        row(p["nm_b2"]))
    return (h_new, x_new)
